# baseline (speedup 1.0000x reference)
import functools

import jax
import jax.numpy as jnp
from jax import lax
from jax.experimental import pallas as pl
from jax.experimental.pallas import tpu as pltpu
from jax.experimental.pallas import tpu_sc as plsc

N_DRUG = 8000
TAIL = 2000
N = 10000
E = 160000
HID = 512
OUT1 = 256
OUT_CH = 128

F32 = jnp.float32
NC = 2
NS = 16
CH = 128
ROWS_PER_TILE = 640
E_PAD = 163840
NCHUNK = E_PAD // CH


def _proj_both_body(x_ref, dw1_ref, db1_ref, dlg_ref, dlb_ref, dw2_ref,
                    db2_ref, cw1t_ref, cb1_ref, clg_ref, clb_ref, cw2_ref,
                    cb2_ref, h2d_ref, dcs_ref, dcq_ref, h2c_ref, ccs_ref,
                    ccq_ref, acc_ref):
    i = pl.program_id(0)
    nsteps = pl.num_programs(0)
    x = x_ref[...]

    h = lax.dot_general(x, dw1_ref[...], (((1,), (1,)), ((), ())),
                        preferred_element_type=F32) + db1_ref[...]
    mu = jnp.mean(h, axis=-1, keepdims=True)
    v = jnp.mean((h - mu) ** 2, axis=-1, keepdims=True)
    h = (h - mu) / jnp.sqrt(v + 1e-5) * dlg_ref[...] + dlb_ref[...]
    h = jnp.maximum(h, 0.0)
    h2 = lax.dot_general(h, dw2_ref[...], (((1,), (1,)), ((), ())),
                         preferred_element_type=F32) + db2_ref[...]
    h2d_ref[...] = h2

    @pl.when(i == 0)
    def _():
        dcs_ref[...] = jnp.zeros_like(dcs_ref)
        dcq_ref[...] = jnp.zeros_like(dcq_ref)
        acc_ref[...] = jnp.zeros_like(acc_ref)

    dcs_ref[...] += jnp.sum(h2, axis=0, keepdims=True)
    dcq_ref[...] += jnp.sum(h2 * h2, axis=0, keepdims=True)

    acc_ref[...] += lax.dot_general(
        x, cw1t_ref[...], (((0,), (0,)), ((), ())),
        preferred_element_type=F32)

    @pl.when(i == nsteps - 1)
    def _():
        hc = acc_ref[...] + cb1_ref[...]
        muc = jnp.mean(hc, axis=-1, keepdims=True)
        vc = jnp.mean((hc - muc) ** 2, axis=-1, keepdims=True)
        hc = (hc - muc) / jnp.sqrt(vc + 1e-5) * clg_ref[...] + clb_ref[...]
        hc = jnp.maximum(hc, 0.0)
        h2c = lax.dot_general(hc, cw2_ref[...], (((1,), (1,)), ((), ())),
                              preferred_element_type=F32) + cb2_ref[...]
        h2c_ref[...] = h2c
        ccs_ref[...] = jnp.sum(h2c, axis=0, keepdims=True)
        ccq_ref[...] = jnp.sum(h2c * h2c, axis=0, keepdims=True)


def _proj_both(feats, dw1, db1, dlg, dlb, dw2, db2,
               cw1t, cb1, clg, clb, cw2, cb2):
    bm = 1000
    grid = (N_DRUG // bm,)
    vec = pl.BlockSpec((1, HID), lambda i: (0, 0))
    return pl.pallas_call(
        _proj_both_body,
        grid=grid,
        in_specs=[
            pl.BlockSpec((bm, TAIL), lambda i: (i, 0)),
            pl.BlockSpec((HID, TAIL), lambda i: (0, 0)),
            vec, vec, vec,
            pl.BlockSpec((HID, HID), lambda i: (0, 0)),
            vec,
            pl.BlockSpec((bm, HID), lambda i: (i, 0)),
            vec, vec, vec,
            pl.BlockSpec((HID, HID), lambda i: (0, 0)),
            vec,
        ],
        out_specs=[
            pl.BlockSpec((bm, HID), lambda i: (i, 0)),
            vec, vec,
            pl.BlockSpec((TAIL, HID), lambda i: (0, 0)),
            vec, vec,
        ],
        out_shape=[
            jax.ShapeDtypeStruct((N_DRUG, HID), F32),
            jax.ShapeDtypeStruct((1, HID), F32),
            jax.ShapeDtypeStruct((1, HID), F32),
            jax.ShapeDtypeStruct((TAIL, HID), F32),
            jax.ShapeDtypeStruct((1, HID), F32),
            jax.ShapeDtypeStruct((1, HID), F32),
        ],
        scratch_shapes=[pltpu.VMEM((TAIL, HID), F32)],
    )(feats, dw1, db1, dlg, dlb, dw2, db2, cw1t, cb1, clg, clb, cw2, cb2)


def _bn_gcn1_body(h2_ref, cs_ref, cq_ref, g_ref, b_ref, p0_ref, p1_ref,
                  w_ref, ya_ref, yb_ref):
    i = pl.program_id(0)
    dom = i >= 8
    nrows = jnp.where(dom, float(TAIL), float(N_DRUG))
    cs = cs_ref[...]
    cq = cq_ref[...]
    mu = jnp.where(dom, cs[1:2, :], cs[0:1, :]) / nrows
    var = jnp.where(dom, cq[1:2, :], cq[0:1, :]) / nrows - mu * mu
    g = jnp.where(dom, g_ref[1:2, :], g_ref[0:1, :])
    b = jnp.where(dom, b_ref[1:2, :], b_ref[0:1, :])
    x = (h2_ref[...] - mu) / jnp.sqrt(var + 1e-5) * g + b
    x = jnp.maximum(x, 0.0)
    xw = lax.dot_general(x, w_ref[...], (((1,), (1,)), ((), ())),
                         preferred_element_type=F32)
    dis = lax.rsqrt(p0_ref[...] + p1_ref[...] + 1.0)
    y = xw * dis
    ya_ref[...] = y[:, :OUT_CH]
    yb_ref[...] = y[:, OUT_CH:]


def _bn_gcn1(h2, cs2, cq2, g2, b2, p0, p1, w):
    bm = 1000
    grid = (N // bm,)
    return pl.pallas_call(
        _bn_gcn1_body,
        grid=grid,
        in_specs=[
            pl.BlockSpec((bm, HID), lambda i: (i, 0)),
            pl.BlockSpec((2, HID), lambda i: (0, 0)),
            pl.BlockSpec((2, HID), lambda i: (0, 0)),
            pl.BlockSpec((2, HID), lambda i: (0, 0)),
            pl.BlockSpec((2, HID), lambda i: (0, 0)),
            pl.BlockSpec((bm, 1), lambda i: (i, 0)),
            pl.BlockSpec((bm, 1), lambda i: (i, 0)),
            pl.BlockSpec((OUT1, HID), lambda i: (0, 0)),
        ],
        out_specs=[
            pl.BlockSpec((bm, OUT_CH), lambda i: (i, 0)),
            pl.BlockSpec((bm, OUT_CH), lambda i: (i, 0)),
        ],
        out_shape=[
            jax.ShapeDtypeStruct((N, OUT_CH), F32),
            jax.ShapeDtypeStruct((N, OUT_CH), F32),
        ],
    )(h2, cs2, cq2, g2, b2, p0, p1, w)


def _gcn2_in_body(aa_ref, ab_ref, ya_ref, yb_ref, p0_ref, p1_ref, b1_ref,
                  w_ref, o_ref):
    dis = lax.rsqrt(p0_ref[...] + p1_ref[...] + 1.0)
    s = jnp.concatenate([aa_ref[...] + ya_ref[...],
                         ab_ref[...] + yb_ref[...]], axis=1)
    x1 = jnp.maximum(dis * s + b1_ref[...], 0.0)
    xw = lax.dot_general(x1, w_ref[...], (((1,), (1,)), ((), ())),
                         preferred_element_type=F32)
    o_ref[...] = xw * dis


def _gcn2_in(aggA, aggB, yA, yB, p0, p1, b1, w):
    bm = 1000
    grid = (N // bm,)
    return pl.pallas_call(
        _gcn2_in_body,
        grid=grid,
        in_specs=[
            pl.BlockSpec((bm, OUT_CH), lambda i: (i, 0)),
            pl.BlockSpec((bm, OUT_CH), lambda i: (i, 0)),
            pl.BlockSpec((bm, OUT_CH), lambda i: (i, 0)),
            pl.BlockSpec((bm, OUT_CH), lambda i: (i, 0)),
            pl.BlockSpec((bm, 1), lambda i: (i, 0)),
            pl.BlockSpec((bm, 1), lambda i: (i, 0)),
            pl.BlockSpec((1, OUT1), lambda i: (0, 0)),
            pl.BlockSpec((OUT_CH, OUT1), lambda i: (0, 0)),
        ],
        out_specs=pl.BlockSpec((bm, OUT_CH), lambda i: (i, 0)),
        out_shape=jax.ShapeDtypeStruct((N, OUT_CH), F32),
    )(aggA, aggB, yA, yB, p0, p1, b1, w)


def _final_body(a0_ref, a1_ref, y2_ref, p0_ref, p1_ref, b2_ref,
                lw_ref, lb_ref, df_ref, z_ref):
    dis = lax.rsqrt(p0_ref[...] + p1_ref[...] + 1.0)
    s = a0_ref[...] + a1_ref[...] + y2_ref[...]
    df = dis * s + b2_ref[...]
    df_ref[...] = df
    z_ref[...] = lax.dot_general(df, lw_ref[...], (((1,), (1,)), ((), ())),
                                 preferred_element_type=F32) + lb_ref[...]


def _final(agg2P0, agg2P1, y2, p0, p1, b2, lw, lb):
    bm = 1000
    grid = (N // bm,)
    return pl.pallas_call(
        _final_body,
        grid=grid,
        in_specs=[
            pl.BlockSpec((bm, OUT_CH), lambda i: (i, 0)),
            pl.BlockSpec((bm, OUT_CH), lambda i: (i, 0)),
            pl.BlockSpec((bm, OUT_CH), lambda i: (i, 0)),
            pl.BlockSpec((bm, 1), lambda i: (i, 0)),
            pl.BlockSpec((bm, 1), lambda i: (i, 0)),
            pl.BlockSpec((1, OUT_CH), lambda i: (0, 0)),
            pl.BlockSpec((OUT_CH, OUT_CH), lambda i: (0, 0)),
            pl.BlockSpec((1, OUT_CH), lambda i: (0, 0)),
        ],
        out_specs=[
            pl.BlockSpec((bm, OUT_CH), lambda i: (i, 0)),
            pl.BlockSpec((bm, OUT_CH), lambda i: (i, 0)),
        ],
        out_shape=[
            jax.ShapeDtypeStruct((N, OUT_CH), F32),
            jax.ShapeDtypeStruct((N, OUT_CH), F32),
        ],
    )(agg2P0, agg2P1, y2, p0, p1, b2, lw, lb)


_MESH = plsc.VectorSubcoreMesh(core_axis_name="c", subcore_axis_name="s")


N_PAD = 10240


def _deg_body(dst_hbm, zer_hbm, one_hbm, out0, out1, acc, idx2, onev, semi):
    c = lax.axis_index("c")
    s = lax.axis_index("s")
    sl = pl.ds(640 * s, 640)
    pltpu.sync_copy(zer_hbm, acc.at[sl])
    pltpu.sync_copy(one_hbm, onev)
    plsc.subcore_barrier()

    nkt = NCHUNK // NC // NS

    def base(k):
        return pl.multiple_of((c * (NCHUNK // NC) + s * nkt + k) * CH, CH)

    pltpu.sync_copy(dst_hbm.at[pl.ds(base(0), CH)], idx2.at[0])

    @pl.loop(0, nkt)
    def _(k):
        pb = lax.rem(k, 2)
        pn = 1 - pb

        @pl.when(k > 0)
        def _():
            pltpu.make_async_copy(dst_hbm.at[pl.ds(base(k), CH)],
                                  idx2.at[pb], semi).wait()

        @pl.when(k < nkt - 1)
        def _():
            pltpu.async_copy(dst_hbm.at[pl.ds(base(k + 1), CH)],
                             idx2.at[pn], semi)

        pltpu.sync_copy(onev, acc.at[idx2.at[pb]], add=True)

    plsc.subcore_barrier()

    @pl.when(c == 0)
    def _():
        pltpu.sync_copy(acc.at[sl], out0.at[sl])

    @pl.when(c == 1)
    def _():
        pltpu.sync_copy(acc.at[sl], out1.at[sl])


def _deg_hist(dst, zeros640, ones128):
    fn = pl.kernel(
        _deg_body,
        out_type=(jax.ShapeDtypeStruct((N_PAD,), F32),
                  jax.ShapeDtypeStruct((N_PAD,), F32)),
        mesh=_MESH,
        scratch_types=[
            pltpu.VMEM_SHARED((N_PAD,), F32),
            pltpu.VMEM((2, CH), jnp.int32),
            pltpu.VMEM((CH,), F32),
            pltpu.SemaphoreType.DMA,
        ],
    )
    return fn(dst, zeros640, ones128)


def _seg_body(ya_hbm, yb_hbm, src_hbm, dst_hbm, zer_hbm, outA, outB,
              acc, idxs2, idxd2, rows2, semg):
    c = lax.axis_index("c")
    s = lax.axis_index("s")
    sl = pl.ds(s * ROWS_PER_TILE, ROWS_PER_TILE)
    pltpu.sync_copy(zer_hbm, acc.at[sl])
    plsc.subcore_barrier()

    nkt = NCHUNK // NS

    def base(k):
        return pl.multiple_of((s * nkt + k) * CH, CH)

    def load_idx(k, b):
        pltpu.sync_copy(src_hbm.at[pl.ds(base(k), CH)], idxs2.at[b])
        pltpu.sync_copy(dst_hbm.at[pl.ds(base(k), CH)], idxd2.at[b])

    def fire_gather(b):
        @pl.when(c == 0)
        def _():
            pltpu.async_copy(ya_hbm.at[idxs2.at[b]], rows2.at[b], semg)

        @pl.when(c == 1)
        def _():
            pltpu.async_copy(yb_hbm.at[idxs2.at[b]], rows2.at[b], semg)

    load_idx(0, 0)
    fire_gather(0)

    @pl.loop(0, nkt)
    def _(k):
        pb = lax.rem(k, 2)
        pn = 1 - pb

        @pl.when(k < nkt - 1)
        def _():
            load_idx(k + 1, pn)
            fire_gather(pn)

        pltpu.make_async_copy(ya_hbm.at[idxs2.at[pb]], rows2.at[pb],
                              semg).wait()
        pltpu.sync_copy(rows2.at[pb], acc.at[idxd2.at[pb]], add=True)

    plsc.subcore_barrier()

    @pl.when(c == 0)
    def _():
        pltpu.sync_copy(acc.at[sl], outA.at[sl])

    @pl.when(c == 1)
    def _():
        pltpu.sync_copy(acc.at[sl], outB.at[sl])


def _seg_sum(yA, yB, src, dst, zeros_rows):
    fn = pl.kernel(
        _seg_body,
        out_type=(jax.ShapeDtypeStruct((N_PAD, OUT_CH), F32),
                  jax.ShapeDtypeStruct((N_PAD, OUT_CH), F32)),
        mesh=_MESH,
        scratch_types=[
            pltpu.VMEM_SHARED((N_PAD, OUT_CH), F32),
            pltpu.VMEM((2, CH), jnp.int32),
            pltpu.VMEM((2, CH), jnp.int32),
            pltpu.VMEM((2, CH, OUT_CH), F32),
            pltpu.SemaphoreType.DMA,
        ],
    )
    return fn(yA, yB, src, dst, zeros_rows)


def _seg_partial_body(y_hbm, src_hbm, dst_hbm, zer_hbm, out0, out1,
                      acc, idxs2, idxd2, rows2, semg):
    c = lax.axis_index("c")
    s = lax.axis_index("s")
    sl = pl.ds(s * ROWS_PER_TILE, ROWS_PER_TILE)
    pltpu.sync_copy(zer_hbm, acc.at[sl])
    plsc.subcore_barrier()

    nkt = NCHUNK // NC // NS

    def base(k):
        return pl.multiple_of((c * (NCHUNK // NC) + s * nkt + k) * CH, CH)

    def load_idx(k, b):
        pltpu.sync_copy(src_hbm.at[pl.ds(base(k), CH)], idxs2.at[b])
        pltpu.sync_copy(dst_hbm.at[pl.ds(base(k), CH)], idxd2.at[b])

    load_idx(0, 0)
    pltpu.async_copy(y_hbm.at[idxs2.at[0]], rows2.at[0], semg)

    @pl.loop(0, nkt)
    def _(k):
        pb = lax.rem(k, 2)
        pn = 1 - pb

        @pl.when(k < nkt - 1)
        def _():
            load_idx(k + 1, pn)
            pltpu.async_copy(y_hbm.at[idxs2.at[pn]], rows2.at[pn], semg)

        pltpu.make_async_copy(y_hbm.at[idxs2.at[pb]], rows2.at[pb],
                              semg).wait()
        pltpu.sync_copy(rows2.at[pb], acc.at[idxd2.at[pb]], add=True)

    plsc.subcore_barrier()

    @pl.when(c == 0)
    def _():
        pltpu.sync_copy(acc.at[sl], out0.at[sl])

    @pl.when(c == 1)
    def _():
        pltpu.sync_copy(acc.at[sl], out1.at[sl])


def _seg_partial(y, src, dst, zeros_rows):
    fn = pl.kernel(
        _seg_partial_body,
        out_type=(jax.ShapeDtypeStruct((N_PAD, OUT_CH), F32),
                  jax.ShapeDtypeStruct((N_PAD, OUT_CH), F32)),
        mesh=_MESH,
        scratch_types=[
            pltpu.VMEM_SHARED((N_PAD, OUT_CH), F32),
            pltpu.VMEM((2, CH), jnp.int32),
            pltpu.VMEM((2, CH), jnp.int32),
            pltpu.VMEM((2, CH, OUT_CH), F32),
            pltpu.SemaphoreType.DMA,
        ],
    )
    return fn(y, src, dst, zeros_rows)


def _edge_body(z_hbm, src_hbm, dst_hbm, out, idxs2, idxd2, buf2, semg, semw):
    c = lax.axis_index("c")
    s = lax.axis_index("s")
    w = s * NC + c
    nkt = NCHUNK // (NC * NS)

    def base(k):
        return pl.multiple_of((w * nkt + k) * CH, CH)

    def load_idx(k, b):
        pltpu.sync_copy(src_hbm.at[pl.ds(base(k), CH)], idxs2.at[b])
        pltpu.sync_copy(dst_hbm.at[pl.ds(base(k), CH)], idxd2.at[b])

    def fire_gathers(b):
        pltpu.async_copy(z_hbm.at[idxs2.at[b]],
                         buf2.at[b, :, pl.ds(0, OUT_CH)], semg)
        pltpu.async_copy(z_hbm.at[idxd2.at[b]],
                         buf2.at[b, :, pl.ds(OUT_CH, OUT_CH)], semg)

    load_idx(0, 0)
    fire_gathers(0)

    @pl.loop(0, nkt)
    def _(k):
        pb = lax.rem(k, 2)
        pn = 1 - pb

        @pl.when(k > 0)
        def _():
            pltpu.make_async_copy(buf2.at[pn],
                                  out.at[pl.ds(base(k - 1), CH)],
                                  semw).wait()

        @pl.when(k < nkt - 1)
        def _():
            load_idx(k + 1, pn)
            fire_gathers(pn)

        pltpu.make_async_copy(z_hbm.at[idxs2.at[pb]],
                              buf2.at[pb, :, pl.ds(0, OUT_CH)], semg).wait()
        pltpu.make_async_copy(z_hbm.at[idxd2.at[pb]],
                              buf2.at[pb, :, pl.ds(OUT_CH, OUT_CH)],
                              semg).wait()
        pltpu.async_copy(buf2.at[pb], out.at[pl.ds(base(k), CH)], semw)

    lastb = (nkt - 1) % 2
    pltpu.make_async_copy(buf2.at[lastb],
                          out.at[pl.ds(base(nkt - 1), CH)], semw).wait()


def _edge_gather(z, src, dst):
    fn = pl.kernel(
        _edge_body,
        out_type=jax.ShapeDtypeStruct((E_PAD, 2 * OUT_CH), F32),
        mesh=_MESH,
        scratch_types=[
            pltpu.VMEM((2, CH), jnp.int32),
            pltpu.VMEM((2, CH), jnp.int32),
            pltpu.VMEM((2, CH, 2 * OUT_CH), F32),
            pltpu.SemaphoreType.DMA,
            pltpu.SemaphoreType.DMA,
        ],
    )
    return fn(z, src, dst)


def kernel(feats, dW1, db1, dlng, dlnb, dW2, db2, dbng, dbnb, cW1, cb1, clng,
           clnb, cW2, cb2, cbng, cbnb, convW1, convb1, convW2, convb2, linW,
           linb, edge_index, idx):
    src = edge_index[0]
    dst = edge_index[1]
    r = lambda v: v.reshape(1, -1)

    npad = E_PAD - E
    src_p = jnp.concatenate([src, jnp.zeros((npad,), jnp.int32)])
    dst_p = jnp.concatenate([dst, jnp.full((npad,), N, jnp.int32)])

    zeros640 = jnp.zeros((640,), F32)
    ones128 = jnp.ones((CH,), F32)
    d0, d1 = _deg_hist(dst_p, zeros640, ones128)
    p0 = d0[:N].reshape(N, 1)
    p1 = d1[:N].reshape(N, 1)

    (h2_d, cs_d, cq_d, h2_c, cs_c, cq_c) = _proj_both(
        feats, dW1, r(db1), r(dlng), r(dlnb), dW2, r(db2),
        cW1.T, r(cb1), r(clng), r(clnb), cW2, r(cb2))

    h2 = jnp.concatenate([h2_d, h2_c], axis=0)
    cs2 = jnp.concatenate([cs_d, cs_c], axis=0)
    cq2 = jnp.concatenate([cq_d, cq_c], axis=0)
    g2 = jnp.stack([dbng, cbng], axis=0)
    b2 = jnp.stack([dbnb, cbnb], axis=0)

    yA, yB = _bn_gcn1(h2, cs2, cq2, g2, b2, p0, p1, convW1)

    zrows128 = jnp.zeros((ROWS_PER_TILE, OUT_CH), F32)
    aggA, aggB = _seg_sum(yA, yB, src_p, dst_p, zrows128)
    aggA, aggB = aggA[:N], aggB[:N]

    y2 = _gcn2_in(aggA, aggB, yA, yB, p0, p1, r(convb1), convW2)

    agg2P0, agg2P1 = _seg_partial(y2, src_p, dst_p, zrows128)
    agg2P0, agg2P1 = agg2P0[:N], agg2P1[:N]

    drug_f, z = _final(agg2P0, agg2P1, y2, p0, p1, r(convb2),
                       linW, r(linb))

    edge_feat = _edge_gather(z, src_p, dst_p)[:E]
    return (drug_f, edge_feat, idx)


# speedup vs baseline: 1.3815x; 1.3815x over previous
import functools

import jax
import jax.numpy as jnp
from jax import lax
from jax.experimental import pallas as pl
from jax.experimental.pallas import tpu as pltpu
from jax.experimental.pallas import tpu_sc as plsc

N_DRUG = 8000
TAIL = 2000
N = 10000
E = 160000
HID = 512
OUT1 = 256
OUT_CH = 128

F32 = jnp.float32
NC = 2
NS = 16
CH = 128
ROWS_PER_TILE = 640
E_PAD = 163840
NCHUNK = E_PAD // CH


def _proj_both_body(x_ref, dw1_ref, db1_ref, dlg_ref, dlb_ref, dw2_ref,
                    db2_ref, cw1t_ref, cb1_ref, clg_ref, clb_ref, cw2_ref,
                    cb2_ref, h2d_ref, dcs_ref, dcq_ref, h2c_ref, ccs_ref,
                    ccq_ref, acc_ref):
    i = pl.program_id(0)
    nsteps = pl.num_programs(0)
    x = x_ref[...]

    h = lax.dot_general(x, dw1_ref[...], (((1,), (1,)), ((), ())),
                        preferred_element_type=F32) + db1_ref[...]
    mu = jnp.mean(h, axis=-1, keepdims=True)
    v = jnp.mean((h - mu) ** 2, axis=-1, keepdims=True)
    h = (h - mu) / jnp.sqrt(v + 1e-5) * dlg_ref[...] + dlb_ref[...]
    h = jnp.maximum(h, 0.0)
    h2 = lax.dot_general(h, dw2_ref[...], (((1,), (1,)), ((), ())),
                         preferred_element_type=F32) + db2_ref[...]
    h2d_ref[...] = h2

    @pl.when(i == 0)
    def _():
        dcs_ref[...] = jnp.zeros_like(dcs_ref)
        dcq_ref[...] = jnp.zeros_like(dcq_ref)
        acc_ref[...] = jnp.zeros_like(acc_ref)

    dcs_ref[...] += jnp.sum(h2, axis=0, keepdims=True)
    dcq_ref[...] += jnp.sum(h2 * h2, axis=0, keepdims=True)

    acc_ref[...] += lax.dot_general(
        x, cw1t_ref[...], (((0,), (0,)), ((), ())),
        preferred_element_type=F32)

    @pl.when(i == nsteps - 1)
    def _():
        hc = acc_ref[...] + cb1_ref[...]
        muc = jnp.mean(hc, axis=-1, keepdims=True)
        vc = jnp.mean((hc - muc) ** 2, axis=-1, keepdims=True)
        hc = (hc - muc) / jnp.sqrt(vc + 1e-5) * clg_ref[...] + clb_ref[...]
        hc = jnp.maximum(hc, 0.0)
        h2c = lax.dot_general(hc, cw2_ref[...], (((1,), (1,)), ((), ())),
                              preferred_element_type=F32) + cb2_ref[...]
        h2c_ref[...] = h2c
        ccs_ref[...] = jnp.sum(h2c, axis=0, keepdims=True)
        ccq_ref[...] = jnp.sum(h2c * h2c, axis=0, keepdims=True)


def _proj_both(feats, dw1, db1, dlg, dlb, dw2, db2,
               cw1t, cb1, clg, clb, cw2, cb2):
    bm = 1000
    grid = (N_DRUG // bm,)
    vec = pl.BlockSpec((1, HID), lambda i: (0, 0))
    return pl.pallas_call(
        _proj_both_body,
        grid=grid,
        in_specs=[
            pl.BlockSpec((bm, TAIL), lambda i: (i, 0)),
            pl.BlockSpec((HID, TAIL), lambda i: (0, 0)),
            vec, vec, vec,
            pl.BlockSpec((HID, HID), lambda i: (0, 0)),
            vec,
            pl.BlockSpec((bm, HID), lambda i: (i, 0)),
            vec, vec, vec,
            pl.BlockSpec((HID, HID), lambda i: (0, 0)),
            vec,
        ],
        out_specs=[
            pl.BlockSpec((bm, HID), lambda i: (i, 0)),
            vec, vec,
            pl.BlockSpec((TAIL, HID), lambda i: (0, 0)),
            vec, vec,
        ],
        out_shape=[
            jax.ShapeDtypeStruct((N_DRUG, HID), F32),
            jax.ShapeDtypeStruct((1, HID), F32),
            jax.ShapeDtypeStruct((1, HID), F32),
            jax.ShapeDtypeStruct((TAIL, HID), F32),
            jax.ShapeDtypeStruct((1, HID), F32),
            jax.ShapeDtypeStruct((1, HID), F32),
        ],
        scratch_shapes=[pltpu.VMEM((TAIL, HID), F32)],
    )(feats, dw1, db1, dlg, dlb, dw2, db2, cw1t, cb1, clg, clb, cw2, cb2)


def _bn_gcn1_body(h2_ref, cs_ref, cq_ref, g_ref, b_ref, p0_ref, p1_ref,
                  w_ref, ya_ref, yb_ref):
    i = pl.program_id(0)
    dom = i >= 8
    nrows = jnp.where(dom, float(TAIL), float(N_DRUG))
    cs = cs_ref[...]
    cq = cq_ref[...]
    mu = jnp.where(dom, cs[1:2, :], cs[0:1, :]) / nrows
    var = jnp.where(dom, cq[1:2, :], cq[0:1, :]) / nrows - mu * mu
    g = jnp.where(dom, g_ref[1:2, :], g_ref[0:1, :])
    b = jnp.where(dom, b_ref[1:2, :], b_ref[0:1, :])
    x = (h2_ref[...] - mu) / jnp.sqrt(var + 1e-5) * g + b
    x = jnp.maximum(x, 0.0)
    xw = lax.dot_general(x, w_ref[...], (((1,), (1,)), ((), ())),
                         preferred_element_type=F32)
    dis = lax.rsqrt(p0_ref[...] + p1_ref[...] + 1.0)
    y = xw * dis
    ya_ref[...] = y[:, :OUT_CH]
    yb_ref[...] = y[:, OUT_CH:]


def _bn_gcn1(h2, cs2, cq2, g2, b2, p0, p1, w):
    bm = 1000
    grid = (N // bm,)
    return pl.pallas_call(
        _bn_gcn1_body,
        grid=grid,
        in_specs=[
            pl.BlockSpec((bm, HID), lambda i: (i, 0)),
            pl.BlockSpec((2, HID), lambda i: (0, 0)),
            pl.BlockSpec((2, HID), lambda i: (0, 0)),
            pl.BlockSpec((2, HID), lambda i: (0, 0)),
            pl.BlockSpec((2, HID), lambda i: (0, 0)),
            pl.BlockSpec((bm, 1), lambda i: (i, 0)),
            pl.BlockSpec((bm, 1), lambda i: (i, 0)),
            pl.BlockSpec((OUT1, HID), lambda i: (0, 0)),
        ],
        out_specs=[
            pl.BlockSpec((bm, OUT_CH), lambda i: (i, 0)),
            pl.BlockSpec((bm, OUT_CH), lambda i: (i, 0)),
        ],
        out_shape=[
            jax.ShapeDtypeStruct((N, OUT_CH), F32),
            jax.ShapeDtypeStruct((N, OUT_CH), F32),
        ],
    )(h2, cs2, cq2, g2, b2, p0, p1, w)


def _gcn2_in_body(aa_ref, ab_ref, ya_ref, yb_ref, p0_ref, p1_ref, b1_ref,
                  w_ref, o_ref):
    dis = lax.rsqrt(p0_ref[...] + p1_ref[...] + 1.0)
    s = jnp.concatenate([aa_ref[...] + ya_ref[...],
                         ab_ref[...] + yb_ref[...]], axis=1)
    x1 = jnp.maximum(dis * s + b1_ref[...], 0.0)
    xw = lax.dot_general(x1, w_ref[...], (((1,), (1,)), ((), ())),
                         preferred_element_type=F32)
    o_ref[...] = xw * dis


def _gcn2_in(aggA, aggB, yA, yB, p0, p1, b1, w):
    bm = 1000
    grid = (N // bm,)
    return pl.pallas_call(
        _gcn2_in_body,
        grid=grid,
        in_specs=[
            pl.BlockSpec((bm, OUT_CH), lambda i: (i, 0)),
            pl.BlockSpec((bm, OUT_CH), lambda i: (i, 0)),
            pl.BlockSpec((bm, OUT_CH), lambda i: (i, 0)),
            pl.BlockSpec((bm, OUT_CH), lambda i: (i, 0)),
            pl.BlockSpec((bm, 1), lambda i: (i, 0)),
            pl.BlockSpec((bm, 1), lambda i: (i, 0)),
            pl.BlockSpec((1, OUT1), lambda i: (0, 0)),
            pl.BlockSpec((OUT_CH, OUT1), lambda i: (0, 0)),
        ],
        out_specs=pl.BlockSpec((bm, OUT_CH), lambda i: (i, 0)),
        out_shape=jax.ShapeDtypeStruct((N, OUT_CH), F32),
    )(aggA, aggB, yA, yB, p0, p1, b1, w)


def _final_body(a0_ref, a1_ref, y2_ref, p0_ref, p1_ref, b2_ref,
                lw_ref, lb_ref, df_ref, z_ref):
    dis = lax.rsqrt(p0_ref[...] + p1_ref[...] + 1.0)
    s = a0_ref[...] + a1_ref[...] + y2_ref[...]
    df = dis * s + b2_ref[...]
    df_ref[...] = df
    z_ref[...] = lax.dot_general(df, lw_ref[...], (((1,), (1,)), ((), ())),
                                 preferred_element_type=F32) + lb_ref[...]


def _final(agg2P0, agg2P1, y2, p0, p1, b2, lw, lb):
    bm = 1000
    grid = (N // bm,)
    return pl.pallas_call(
        _final_body,
        grid=grid,
        in_specs=[
            pl.BlockSpec((bm, OUT_CH), lambda i: (i, 0)),
            pl.BlockSpec((bm, OUT_CH), lambda i: (i, 0)),
            pl.BlockSpec((bm, OUT_CH), lambda i: (i, 0)),
            pl.BlockSpec((bm, 1), lambda i: (i, 0)),
            pl.BlockSpec((bm, 1), lambda i: (i, 0)),
            pl.BlockSpec((1, OUT_CH), lambda i: (0, 0)),
            pl.BlockSpec((OUT_CH, OUT_CH), lambda i: (0, 0)),
            pl.BlockSpec((1, OUT_CH), lambda i: (0, 0)),
        ],
        out_specs=[
            pl.BlockSpec((bm, OUT_CH), lambda i: (i, 0)),
            pl.BlockSpec((bm, OUT_CH), lambda i: (i, 0)),
        ],
        out_shape=[
            jax.ShapeDtypeStruct((N, OUT_CH), F32),
            jax.ShapeDtypeStruct((N, OUT_CH), F32),
        ],
    )(agg2P0, agg2P1, y2, p0, p1, b2, lw, lb)


_MESH = plsc.VectorSubcoreMesh(core_axis_name="c", subcore_axis_name="s")


N_PAD = 10240


def _deg_body(dst_hbm, zer_hbm, one_hbm, out0, out1, acc, idxa, onev, sema):
    c = lax.axis_index("c")
    s = lax.axis_index("s")
    sl = pl.ds(640 * s, 640)
    nkt = NCHUNK // NC // NS
    row0 = c * (NCHUNK // NC) + s * nkt
    pltpu.sync_copy(zer_hbm, acc.at[sl])
    pltpu.sync_copy(one_hbm, onev)
    pltpu.sync_copy(dst_hbm.at[pl.ds(row0, nkt)], idxa)
    plsc.subcore_barrier()

    @pl.loop(0, nkt)
    def _(k):
        pltpu.async_copy(onev, acc.at[idxa.at[k]], sema, add=True)

        @pl.when(k >= 3)
        def _():
            pltpu.make_async_copy(onev, acc.at[idxa.at[k - 3]], sema).wait()

    for t in range(3):
        pltpu.make_async_copy(onev, acc.at[idxa.at[nkt - 3 + t]], sema).wait()

    plsc.subcore_barrier()

    @pl.when(c == 0)
    def _():
        pltpu.sync_copy(acc.at[sl], out0.at[sl])

    @pl.when(c == 1)
    def _():
        pltpu.sync_copy(acc.at[sl], out1.at[sl])


def _deg_hist(dst2d, zeros640, ones128):
    nkt = NCHUNK // NC // NS
    fn = pl.kernel(
        _deg_body,
        out_type=(jax.ShapeDtypeStruct((N_PAD,), F32),
                  jax.ShapeDtypeStruct((N_PAD,), F32)),
        mesh=_MESH,
        scratch_types=[
            pltpu.VMEM_SHARED((N_PAD,), F32),
            pltpu.VMEM((nkt, CH), jnp.int32),
            pltpu.VMEM((CH,), F32),
            pltpu.SemaphoreType.DMA,
        ],
    )
    return fn(dst2d, zeros640, ones128)


def _make_seg_sum_body(split_cols, nkt, row0_fn):
    def body(ya_hbm, yb_hbm, src_hbm, dst_hbm, zer_hbm, outA, outB,
             acc, idxs2, idxd2, rows2, semg, sema):
        c = lax.axis_index("c")
        s = lax.axis_index("s")
        sl = pl.ds(s * ROWS_PER_TILE, ROWS_PER_TILE)
        row0 = row0_fn(c, s)
        pltpu.sync_copy(zer_hbm, acc.at[sl])
        plsc.subcore_barrier()

        def load_idx(k, b):
            pltpu.sync_copy(src_hbm.at[row0 + k], idxs2.at[b])
            pltpu.sync_copy(dst_hbm.at[row0 + k], idxd2.at[b])

        def fire_gather(b):
            if split_cols:
                @pl.when(c == 0)
                def _():
                    pltpu.async_copy(ya_hbm.at[idxs2.at[b]],
                                     rows2.at[b], semg)

                @pl.when(c == 1)
                def _():
                    pltpu.async_copy(yb_hbm.at[idxs2.at[b]],
                                     rows2.at[b], semg)
            else:
                pltpu.async_copy(ya_hbm.at[idxs2.at[b]], rows2.at[b], semg)

        load_idx(0, 0)
        fire_gather(0)

        @pl.loop(0, nkt)
        def _(k):
            pb = lax.rem(k, 2)
            pn = 1 - pb

            @pl.when(k > 0)
            def _():
                pltpu.make_async_copy(rows2.at[pn], acc.at[idxd2.at[pn]],
                                      sema).wait()

            @pl.when(k < nkt - 1)
            def _():
                load_idx(k + 1, pn)
                fire_gather(pn)

            pltpu.make_async_copy(ya_hbm.at[idxs2.at[pb]], rows2.at[pb],
                                  semg).wait()
            pltpu.async_copy(rows2.at[pb], acc.at[idxd2.at[pb]], sema,
                             add=True)

        lastb = (nkt - 1) % 2
        pltpu.make_async_copy(rows2.at[lastb], acc.at[idxd2.at[lastb]],
                              sema).wait()
        plsc.subcore_barrier()

        @pl.when(c == 0)
        def _():
            pltpu.sync_copy(acc.at[sl], outA.at[sl])

        @pl.when(c == 1)
        def _():
            pltpu.sync_copy(acc.at[sl], outB.at[sl])

    return body


def _seg_scratch(nkt):
    return [
        pltpu.VMEM_SHARED((N_PAD, OUT_CH), F32),
        pltpu.VMEM((2, CH), jnp.int32),
        pltpu.VMEM((2, CH), jnp.int32),
        pltpu.VMEM((2, CH, OUT_CH), F32),
        pltpu.SemaphoreType.DMA,
        pltpu.SemaphoreType.DMA,
    ]


def _seg_sum(yA, yB, src2d, dst2d, zeros_rows):
    nkt = NCHUNK // NS
    body = _make_seg_sum_body(True, nkt, lambda c, s: s * nkt)
    fn = pl.kernel(
        body,
        out_type=(jax.ShapeDtypeStruct((N_PAD, OUT_CH), F32),
                  jax.ShapeDtypeStruct((N_PAD, OUT_CH), F32)),
        mesh=_MESH,
        scratch_types=_seg_scratch(nkt),
    )
    return fn(yA, yB, src2d, dst2d, zeros_rows)


def _seg_partial(y, src2d, dst2d, zeros_rows):
    nkt = NCHUNK // NC // NS
    body = _make_seg_sum_body(
        False, nkt, lambda c, s: c * (NCHUNK // NC) + s * nkt)
    fn = pl.kernel(
        body,
        out_type=(jax.ShapeDtypeStruct((N_PAD, OUT_CH), F32),
                  jax.ShapeDtypeStruct((N_PAD, OUT_CH), F32)),
        mesh=_MESH,
        scratch_types=_seg_scratch(nkt),
    )
    return fn(y, y, src2d, dst2d, zeros_rows)


def _edge_body(z_hbm, src_hbm, dst_hbm, out, idxsa, idxda, buf3, semg, semw):
    c = lax.axis_index("c")
    s = lax.axis_index("s")
    w = s * NC + c
    nkt = NCHUNK // (NC * NS)
    row0 = w * nkt
    pltpu.sync_copy(src_hbm.at[pl.ds(row0, nkt)], idxsa)
    pltpu.sync_copy(dst_hbm.at[pl.ds(row0, nkt)], idxda)

    def base(k):
        return pl.multiple_of((row0 + k) * CH, CH)

    def fire_gathers(k):
        slot = lax.rem(k, 3)
        pltpu.async_copy(z_hbm.at[idxsa.at[k]],
                         buf3.at[slot, :, pl.ds(0, OUT_CH)], semg)
        pltpu.async_copy(z_hbm.at[idxda.at[k]],
                         buf3.at[slot, :, pl.ds(OUT_CH, OUT_CH)], semg)

    fire_gathers(0)
    fire_gathers(1)

    @pl.loop(0, nkt)
    def _(k):
        slot = lax.rem(k, 3)

        @pl.when(k > 0)
        def _():
            pltpu.make_async_copy(buf3.at[lax.rem(k + 2, 3)],
                                  out.at[pl.ds(base(k - 1), CH)],
                                  semw).wait()

        @pl.when(k < nkt - 2)
        def _():
            fire_gathers(k + 2)

        pltpu.make_async_copy(z_hbm.at[idxsa.at[k]],
                              buf3.at[slot, :, pl.ds(0, OUT_CH)],
                              semg).wait()
        pltpu.make_async_copy(z_hbm.at[idxda.at[k]],
                              buf3.at[slot, :, pl.ds(OUT_CH, OUT_CH)],
                              semg).wait()
        pltpu.async_copy(buf3.at[slot], out.at[pl.ds(base(k), CH)], semw)

    pltpu.make_async_copy(buf3.at[lax.rem(nkt - 1, 3)],
                          out.at[pl.ds(base(nkt - 1), CH)], semw).wait()


def _edge_gather(z, src2d, dst2d):
    nkt = NCHUNK // (NC * NS)
    fn = pl.kernel(
        _edge_body,
        out_type=jax.ShapeDtypeStruct((E_PAD, 2 * OUT_CH), F32),
        mesh=_MESH,
        scratch_types=[
            pltpu.VMEM((nkt, CH), jnp.int32),
            pltpu.VMEM((nkt, CH), jnp.int32),
            pltpu.VMEM((3, CH, 2 * OUT_CH), F32),
            pltpu.SemaphoreType.DMA,
            pltpu.SemaphoreType.DMA,
        ],
    )
    return fn(z, src2d, dst2d)


def kernel(feats, dW1, db1, dlng, dlnb, dW2, db2, dbng, dbnb, cW1, cb1, clng,
           clnb, cW2, cb2, cbng, cbnb, convW1, convb1, convW2, convb2, linW,
           linb, edge_index, idx):
    src = edge_index[0]
    dst = edge_index[1]
    r = lambda v: v.reshape(1, -1)

    npad = E_PAD - E
    src_p = jnp.concatenate([src, jnp.zeros((npad,), jnp.int32)])
    dst_p = jnp.concatenate([dst, jnp.full((npad,), N, jnp.int32)])
    src2d = src_p.reshape(NCHUNK, CH)
    dst2d = dst_p.reshape(NCHUNK, CH)

    zeros640 = jnp.zeros((640,), F32)
    ones128 = jnp.ones((CH,), F32)
    d0, d1 = _deg_hist(dst2d, zeros640, ones128)
    p0 = d0[:N].reshape(N, 1)
    p1 = d1[:N].reshape(N, 1)

    (h2_d, cs_d, cq_d, h2_c, cs_c, cq_c) = _proj_both(
        feats, dW1, r(db1), r(dlng), r(dlnb), dW2, r(db2),
        cW1.T, r(cb1), r(clng), r(clnb), cW2, r(cb2))

    h2 = jnp.concatenate([h2_d, h2_c], axis=0)
    cs2 = jnp.concatenate([cs_d, cs_c], axis=0)
    cq2 = jnp.concatenate([cq_d, cq_c], axis=0)
    g2 = jnp.stack([dbng, cbng], axis=0)
    b2 = jnp.stack([dbnb, cbnb], axis=0)

    yA, yB = _bn_gcn1(h2, cs2, cq2, g2, b2, p0, p1, convW1)

    zrows128 = jnp.zeros((ROWS_PER_TILE, OUT_CH), F32)
    aggA, aggB = _seg_sum(yA, yB, src2d, dst2d, zrows128)
    aggA, aggB = aggA[:N], aggB[:N]

    y2 = _gcn2_in(aggA, aggB, yA, yB, p0, p1, r(convb1), convW2)

    agg2P0, agg2P1 = _seg_partial(y2, src2d, dst2d, zrows128)
    agg2P0, agg2P1 = agg2P0[:N], agg2P1[:N]

    drug_f, z = _final(agg2P0, agg2P1, y2, p0, p1, r(convb2),
                       linW, r(linb))

    edge_feat = _edge_gather(z, src2d, dst2d)[:E]
    return (drug_f, edge_feat, idx)


# speedup vs baseline: 1.4249x; 1.0314x over previous
import functools

import jax
import jax.numpy as jnp
from jax import lax
from jax.experimental import pallas as pl
from jax.experimental.pallas import tpu as pltpu
from jax.experimental.pallas import tpu_sc as plsc

N_DRUG = 8000
TAIL = 2000
N = 10000
E = 160000
HID = 512
OUT1 = 256
OUT_CH = 128

F32 = jnp.float32
NC = 2
NS = 16
CH = 128
ROWS_PER_TILE = 640
E_PAD = 163840
NCHUNK = E_PAD // CH


def _proj_both_body(x_ref, dw1_ref, db1_ref, dlg_ref, dlb_ref, dw2_ref,
                    db2_ref, cw1t_ref, cb1_ref, clg_ref, clb_ref, cw2_ref,
                    cb2_ref, h2d_ref, dcs_ref, dcq_ref, h2c_ref, ccs_ref,
                    ccq_ref, acc_ref):
    i = pl.program_id(0)
    nsteps = pl.num_programs(0)
    x = x_ref[...]

    h = lax.dot_general(x, dw1_ref[...], (((1,), (1,)), ((), ())),
                        preferred_element_type=F32) + db1_ref[...]
    mu = jnp.mean(h, axis=-1, keepdims=True)
    v = jnp.mean((h - mu) ** 2, axis=-1, keepdims=True)
    h = (h - mu) / jnp.sqrt(v + 1e-5) * dlg_ref[...] + dlb_ref[...]
    h = jnp.maximum(h, 0.0)
    h2 = lax.dot_general(h, dw2_ref[...], (((1,), (1,)), ((), ())),
                         preferred_element_type=F32) + db2_ref[...]
    h2d_ref[...] = h2

    @pl.when(i == 0)
    def _():
        dcs_ref[...] = jnp.zeros_like(dcs_ref)
        dcq_ref[...] = jnp.zeros_like(dcq_ref)
        acc_ref[...] = jnp.zeros_like(acc_ref)

    dcs_ref[...] += jnp.sum(h2, axis=0, keepdims=True)
    dcq_ref[...] += jnp.sum(h2 * h2, axis=0, keepdims=True)

    acc_ref[...] += lax.dot_general(
        x, cw1t_ref[...], (((0,), (0,)), ((), ())),
        preferred_element_type=F32)

    @pl.when(i == nsteps - 1)
    def _():
        hc = acc_ref[...] + cb1_ref[...]
        muc = jnp.mean(hc, axis=-1, keepdims=True)
        vc = jnp.mean((hc - muc) ** 2, axis=-1, keepdims=True)
        hc = (hc - muc) / jnp.sqrt(vc + 1e-5) * clg_ref[...] + clb_ref[...]
        hc = jnp.maximum(hc, 0.0)
        h2c = lax.dot_general(hc, cw2_ref[...], (((1,), (1,)), ((), ())),
                              preferred_element_type=F32) + cb2_ref[...]
        h2c_ref[...] = h2c
        ccs_ref[...] = jnp.sum(h2c, axis=0, keepdims=True)
        ccq_ref[...] = jnp.sum(h2c * h2c, axis=0, keepdims=True)


def _proj_both(feats, dw1, db1, dlg, dlb, dw2, db2,
               cw1t, cb1, clg, clb, cw2, cb2):
    bm = 1000
    grid = (N_DRUG // bm,)
    vec = pl.BlockSpec((1, HID), lambda i: (0, 0))
    return pl.pallas_call(
        _proj_both_body,
        grid=grid,
        in_specs=[
            pl.BlockSpec((bm, TAIL), lambda i: (i, 0)),
            pl.BlockSpec((HID, TAIL), lambda i: (0, 0)),
            vec, vec, vec,
            pl.BlockSpec((HID, HID), lambda i: (0, 0)),
            vec,
            pl.BlockSpec((bm, HID), lambda i: (i, 0)),
            vec, vec, vec,
            pl.BlockSpec((HID, HID), lambda i: (0, 0)),
            vec,
        ],
        out_specs=[
            pl.BlockSpec((bm, HID), lambda i: (i, 0)),
            vec, vec,
            pl.BlockSpec((TAIL, HID), lambda i: (0, 0)),
            vec, vec,
        ],
        out_shape=[
            jax.ShapeDtypeStruct((N_DRUG, HID), F32),
            jax.ShapeDtypeStruct((1, HID), F32),
            jax.ShapeDtypeStruct((1, HID), F32),
            jax.ShapeDtypeStruct((TAIL, HID), F32),
            jax.ShapeDtypeStruct((1, HID), F32),
            jax.ShapeDtypeStruct((1, HID), F32),
        ],
        scratch_shapes=[pltpu.VMEM((TAIL, HID), F32)],
    )(feats, dw1, db1, dlg, dlb, dw2, db2, cw1t, cb1, clg, clb, cw2, cb2)


def _bn_gcn1_body(h2_ref, cs_ref, cq_ref, g_ref, b_ref, p0_ref, p1_ref,
                  w_ref, ya_ref, yb_ref):
    i = pl.program_id(0)
    dom = i >= 8
    nrows = jnp.where(dom, float(TAIL), float(N_DRUG))
    cs = cs_ref[...]
    cq = cq_ref[...]
    mu = jnp.where(dom, cs[1:2, :], cs[0:1, :]) / nrows
    var = jnp.where(dom, cq[1:2, :], cq[0:1, :]) / nrows - mu * mu
    g = jnp.where(dom, g_ref[1:2, :], g_ref[0:1, :])
    b = jnp.where(dom, b_ref[1:2, :], b_ref[0:1, :])
    x = (h2_ref[...] - mu) / jnp.sqrt(var + 1e-5) * g + b
    x = jnp.maximum(x, 0.0)
    xw = lax.dot_general(x, w_ref[...], (((1,), (1,)), ((), ())),
                         preferred_element_type=F32)
    dis = lax.rsqrt(p0_ref[...] + p1_ref[...] + 1.0)
    y = xw * dis
    ya_ref[...] = y[:, :OUT_CH]
    yb_ref[...] = y[:, OUT_CH:]


def _bn_gcn1(h2, cs2, cq2, g2, b2, p0, p1, w):
    bm = 1000
    grid = (N // bm,)
    return pl.pallas_call(
        _bn_gcn1_body,
        grid=grid,
        in_specs=[
            pl.BlockSpec((bm, HID), lambda i: (i, 0)),
            pl.BlockSpec((2, HID), lambda i: (0, 0)),
            pl.BlockSpec((2, HID), lambda i: (0, 0)),
            pl.BlockSpec((2, HID), lambda i: (0, 0)),
            pl.BlockSpec((2, HID), lambda i: (0, 0)),
            pl.BlockSpec((bm, 1), lambda i: (i, 0)),
            pl.BlockSpec((bm, 1), lambda i: (i, 0)),
            pl.BlockSpec((OUT1, HID), lambda i: (0, 0)),
        ],
        out_specs=[
            pl.BlockSpec((bm, OUT_CH), lambda i: (i, 0)),
            pl.BlockSpec((bm, OUT_CH), lambda i: (i, 0)),
        ],
        out_shape=[
            jax.ShapeDtypeStruct((N, OUT_CH), F32),
            jax.ShapeDtypeStruct((N, OUT_CH), F32),
        ],
    )(h2, cs2, cq2, g2, b2, p0, p1, w)


def _gcn2_in_body(aa_ref, ab_ref, ya_ref, yb_ref, p0_ref, p1_ref, b1_ref,
                  w_ref, o_ref):
    dis = lax.rsqrt(p0_ref[...] + p1_ref[...] + 1.0)
    s = jnp.concatenate([aa_ref[...] + ya_ref[...],
                         ab_ref[...] + yb_ref[...]], axis=1)
    x1 = jnp.maximum(dis * s + b1_ref[...], 0.0)
    xw = lax.dot_general(x1, w_ref[...], (((1,), (1,)), ((), ())),
                         preferred_element_type=F32)
    o_ref[...] = xw * dis


def _gcn2_in(aggA, aggB, yA, yB, p0, p1, b1, w):
    bm = 1000
    grid = (N // bm,)
    return pl.pallas_call(
        _gcn2_in_body,
        grid=grid,
        in_specs=[
            pl.BlockSpec((bm, OUT_CH), lambda i: (i, 0)),
            pl.BlockSpec((bm, OUT_CH), lambda i: (i, 0)),
            pl.BlockSpec((bm, OUT_CH), lambda i: (i, 0)),
            pl.BlockSpec((bm, OUT_CH), lambda i: (i, 0)),
            pl.BlockSpec((bm, 1), lambda i: (i, 0)),
            pl.BlockSpec((bm, 1), lambda i: (i, 0)),
            pl.BlockSpec((1, OUT1), lambda i: (0, 0)),
            pl.BlockSpec((OUT_CH, OUT1), lambda i: (0, 0)),
        ],
        out_specs=pl.BlockSpec((bm, OUT_CH), lambda i: (i, 0)),
        out_shape=jax.ShapeDtypeStruct((N, OUT_CH), F32),
    )(aggA, aggB, yA, yB, p0, p1, b1, w)


def _final_body(a0_ref, a1_ref, y2_ref, p0_ref, p1_ref, b2_ref,
                lw_ref, lb_ref, df_ref, z_ref):
    dis = lax.rsqrt(p0_ref[...] + p1_ref[...] + 1.0)
    s = a0_ref[...] + a1_ref[...] + y2_ref[...]
    df = dis * s + b2_ref[...]
    df_ref[...] = df
    z_ref[...] = lax.dot_general(df, lw_ref[...], (((1,), (1,)), ((), ())),
                                 preferred_element_type=F32) + lb_ref[...]


def _final(agg2P0, agg2P1, y2, p0, p1, b2, lw, lb):
    bm = 1000
    grid = (N // bm,)
    return pl.pallas_call(
        _final_body,
        grid=grid,
        in_specs=[
            pl.BlockSpec((bm, OUT_CH), lambda i: (i, 0)),
            pl.BlockSpec((bm, OUT_CH), lambda i: (i, 0)),
            pl.BlockSpec((bm, OUT_CH), lambda i: (i, 0)),
            pl.BlockSpec((bm, 1), lambda i: (i, 0)),
            pl.BlockSpec((bm, 1), lambda i: (i, 0)),
            pl.BlockSpec((1, OUT_CH), lambda i: (0, 0)),
            pl.BlockSpec((OUT_CH, OUT_CH), lambda i: (0, 0)),
            pl.BlockSpec((1, OUT_CH), lambda i: (0, 0)),
        ],
        out_specs=[
            pl.BlockSpec((bm, OUT_CH), lambda i: (i, 0)),
            pl.BlockSpec((bm, OUT_CH), lambda i: (i, 0)),
        ],
        out_shape=[
            jax.ShapeDtypeStruct((N, OUT_CH), F32),
            jax.ShapeDtypeStruct((N, OUT_CH), F32),
        ],
    )(agg2P0, agg2P1, y2, p0, p1, b2, lw, lb)


_MESH = plsc.VectorSubcoreMesh(core_axis_name="c", subcore_axis_name="s")


N_PAD = 10240


def _deg_body(dst_hbm, zer_hbm, one_hbm, out0, out1, acc, idxa, onev, sema):
    c = lax.axis_index("c")
    s = lax.axis_index("s")
    sl = pl.ds(640 * s, 640)
    nkt = NCHUNK // NC // NS
    row0 = c * (NCHUNK // NC) + s * nkt
    pltpu.sync_copy(zer_hbm, acc.at[sl])
    pltpu.sync_copy(one_hbm, onev)
    pltpu.sync_copy(dst_hbm.at[pl.ds(row0, nkt)], idxa)
    plsc.subcore_barrier()

    @pl.loop(0, nkt)
    def _(k):
        pltpu.async_copy(onev, acc.at[idxa.at[k]], sema, add=True)

        @pl.when(k >= 3)
        def _():
            pltpu.make_async_copy(onev, acc.at[idxa.at[k - 3]], sema).wait()

    for t in range(3):
        pltpu.make_async_copy(onev, acc.at[idxa.at[nkt - 3 + t]], sema).wait()

    plsc.subcore_barrier()

    @pl.when(c == 0)
    def _():
        pltpu.sync_copy(acc.at[sl], out0.at[sl])

    @pl.when(c == 1)
    def _():
        pltpu.sync_copy(acc.at[sl], out1.at[sl])


def _deg_hist(dst2d, zeros640, ones128):
    nkt = NCHUNK // NC // NS
    fn = pl.kernel(
        _deg_body,
        out_type=(jax.ShapeDtypeStruct((N_PAD,), F32),
                  jax.ShapeDtypeStruct((N_PAD,), F32)),
        mesh=_MESH,
        scratch_types=[
            pltpu.VMEM_SHARED((N_PAD,), F32),
            pltpu.VMEM((nkt, CH), jnp.int32),
            pltpu.VMEM((CH,), F32),
            pltpu.SemaphoreType.DMA,
        ],
    )
    return fn(dst2d, zeros640, ones128)


PH = 40


def _make_seg_sum_body(split_cols, nphase, row0_fn):
    def body(ya_hbm, yb_hbm, src_hbm, dst_hbm, zer_hbm, outA, outB,
             acc, idxsa, idxda, rows2, semg, sema):
        c = lax.axis_index("c")
        s = lax.axis_index("s")
        sl = pl.ds(s * ROWS_PER_TILE, ROWS_PER_TILE)
        pltpu.sync_copy(zer_hbm, acc.at[sl])
        plsc.subcore_barrier()

        def fire_gather(k):
            b = lax.rem(k, 2)
            if split_cols:
                @pl.when(c == 0)
                def _():
                    pltpu.async_copy(ya_hbm.at[idxsa.at[k]],
                                     rows2.at[b], semg)

                @pl.when(c == 1)
                def _():
                    pltpu.async_copy(yb_hbm.at[idxsa.at[k]],
                                     rows2.at[b], semg)
            else:
                pltpu.async_copy(ya_hbm.at[idxsa.at[k]], rows2.at[b], semg)

        for h in range(nphase):
            row0 = row0_fn(c, s) + h * PH
            pltpu.sync_copy(src_hbm.at[pl.ds(row0, PH)], idxsa)
            pltpu.sync_copy(dst_hbm.at[pl.ds(row0, PH)], idxda)
            fire_gather(0)

            @pl.loop(0, PH)
            def _(k):
                pb = lax.rem(k, 2)
                pn = 1 - pb

                @pl.when(k > 0)
                def _():
                    pltpu.make_async_copy(rows2.at[pn],
                                          acc.at[idxda.at[k - 1]],
                                          sema).wait()

                @pl.when(k < PH - 1)
                def _():
                    fire_gather(k + 1)

                pltpu.make_async_copy(ya_hbm.at[idxsa.at[k]], rows2.at[pb],
                                      semg).wait()
                pltpu.async_copy(rows2.at[pb], acc.at[idxda.at[k]], sema,
                                 add=True)

            pltpu.make_async_copy(rows2.at[(PH - 1) % 2],
                                  acc.at[idxda.at[PH - 1]], sema).wait()

        plsc.subcore_barrier()

        @pl.when(c == 0)
        def _():
            pltpu.sync_copy(acc.at[sl], outA.at[sl])

        @pl.when(c == 1)
        def _():
            pltpu.sync_copy(acc.at[sl], outB.at[sl])

    return body


def _seg_scratch():
    return [
        pltpu.VMEM_SHARED((N_PAD, OUT_CH), F32),
        pltpu.VMEM((PH, CH), jnp.int32),
        pltpu.VMEM((PH, CH), jnp.int32),
        pltpu.VMEM((2, CH, OUT_CH), F32),
        pltpu.SemaphoreType.DMA,
        pltpu.SemaphoreType.DMA,
    ]


def _seg_sum(yA, yB, src2d, dst2d, zeros_rows):
    body = _make_seg_sum_body(True, 2, lambda c, s: s * (2 * PH))
    fn = pl.kernel(
        body,
        out_type=(jax.ShapeDtypeStruct((N_PAD, OUT_CH), F32),
                  jax.ShapeDtypeStruct((N_PAD, OUT_CH), F32)),
        mesh=_MESH,
        scratch_types=_seg_scratch(),
    )
    return fn(yA, yB, src2d, dst2d, zeros_rows)


def _seg_partial(y, src2d, dst2d, zeros_rows):
    body = _make_seg_sum_body(
        False, 1, lambda c, s: c * (NCHUNK // NC) + s * PH)
    fn = pl.kernel(
        body,
        out_type=(jax.ShapeDtypeStruct((N_PAD, OUT_CH), F32),
                  jax.ShapeDtypeStruct((N_PAD, OUT_CH), F32)),
        mesh=_MESH,
        scratch_types=_seg_scratch(),
    )
    return fn(y, y, src2d, dst2d, zeros_rows)


def _edge_body(z_hbm, src_hbm, dst_hbm, out, idxsa, idxda, buf3, semg, semw):
    c = lax.axis_index("c")
    s = lax.axis_index("s")
    w = s * NC + c
    nkt = jnp.where(w == NC * NS - 1, 1250 - 40 * (NC * NS - 1), 40)
    row0 = 40 * w
    pltpu.sync_copy(src_hbm.at[pl.ds(row0, 40)], idxsa)
    pltpu.sync_copy(dst_hbm.at[pl.ds(row0, 40)], idxda)

    def base(k):
        return pl.multiple_of((row0 + k) * CH, CH)

    def fire_gathers(k):
        slot = lax.rem(k, 3)
        pltpu.async_copy(z_hbm.at[idxsa.at[k]],
                         buf3.at[slot, :, pl.ds(0, OUT_CH)], semg)
        pltpu.async_copy(z_hbm.at[idxda.at[k]],
                         buf3.at[slot, :, pl.ds(OUT_CH, OUT_CH)], semg)

    fire_gathers(0)
    fire_gathers(1)

    @pl.loop(0, nkt)
    def _(k):
        slot = lax.rem(k, 3)

        @pl.when(k > 0)
        def _():
            pltpu.make_async_copy(buf3.at[lax.rem(k + 2, 3)],
                                  out.at[pl.ds(base(k - 1), CH)],
                                  semw).wait()

        @pl.when(k < nkt - 2)
        def _():
            fire_gathers(k + 2)

        pltpu.make_async_copy(z_hbm.at[idxsa.at[k]],
                              buf3.at[slot, :, pl.ds(0, OUT_CH)],
                              semg).wait()
        pltpu.make_async_copy(z_hbm.at[idxda.at[k]],
                              buf3.at[slot, :, pl.ds(OUT_CH, OUT_CH)],
                              semg).wait()
        pltpu.async_copy(buf3.at[slot], out.at[pl.ds(base(k), CH)], semw)

    pltpu.make_async_copy(buf3.at[lax.rem(nkt - 1, 3)],
                          out.at[pl.ds(base(nkt - 1), CH)], semw).wait()


def _edge_gather(z, src2d, dst2d):
    fn = pl.kernel(
        _edge_body,
        out_type=jax.ShapeDtypeStruct((E, 2 * OUT_CH), F32),
        mesh=_MESH,
        scratch_types=[
            pltpu.VMEM((40, CH), jnp.int32),
            pltpu.VMEM((40, CH), jnp.int32),
            pltpu.VMEM((3, CH, 2 * OUT_CH), F32),
            pltpu.SemaphoreType.DMA,
            pltpu.SemaphoreType.DMA,
        ],
    )
    return fn(z, src2d, dst2d)


def kernel(feats, dW1, db1, dlng, dlnb, dW2, db2, dbng, dbnb, cW1, cb1, clng,
           clnb, cW2, cb2, cbng, cbnb, convW1, convb1, convW2, convb2, linW,
           linb, edge_index, idx):
    src = edge_index[0]
    dst = edge_index[1]
    r = lambda v: v.reshape(1, -1)

    npad = E_PAD - E
    src_p = jnp.concatenate([src, jnp.zeros((npad,), jnp.int32)])
    pad_dst = N + (jnp.arange(npad, dtype=jnp.int32) % (N_PAD - N))
    dst_p = jnp.concatenate([dst, pad_dst])
    src2d = src_p.reshape(NCHUNK, CH)
    dst2d = dst_p.reshape(NCHUNK, CH)

    zeros640 = jnp.zeros((640,), F32)
    ones128 = jnp.ones((CH,), F32)
    d0, d1 = _deg_hist(dst2d, zeros640, ones128)
    p0 = d0.reshape(N_PAD, 1)
    p1 = d1.reshape(N_PAD, 1)

    (h2_d, cs_d, cq_d, h2_c, cs_c, cq_c) = _proj_both(
        feats, dW1, r(db1), r(dlng), r(dlnb), dW2, r(db2),
        cW1.T, r(cb1), r(clng), r(clnb), cW2, r(cb2))

    h2 = jnp.concatenate([h2_d, h2_c], axis=0)
    cs2 = jnp.concatenate([cs_d, cs_c], axis=0)
    cq2 = jnp.concatenate([cq_d, cq_c], axis=0)
    g2 = jnp.stack([dbng, cbng], axis=0)
    b2 = jnp.stack([dbnb, cbnb], axis=0)

    yA, yB = _bn_gcn1(h2, cs2, cq2, g2, b2, p0, p1, convW1)

    zrows128 = jnp.zeros((ROWS_PER_TILE, OUT_CH), F32)
    aggA, aggB = _seg_sum(yA, yB, src2d, dst2d, zrows128)

    y2 = _gcn2_in(aggA, aggB, yA, yB, p0, p1, r(convb1), convW2)

    agg2P0, agg2P1 = _seg_partial(y2, src2d, dst2d, zrows128)

    drug_f, z = _final(agg2P0, agg2P1, y2, p0, p1, r(convb2),
                       linW, r(linb))

    edge_feat = _edge_gather(z, src2d, dst2d)
    return (drug_f, edge_feat, idx)


# speedup vs baseline: 1.5426x; 1.0826x over previous
import functools

import jax
import jax.numpy as jnp
from jax import lax
from jax.experimental import pallas as pl
from jax.experimental.pallas import tpu as pltpu
from jax.experimental.pallas import tpu_sc as plsc

N_DRUG = 8000
TAIL = 2000
N = 10000
E = 160000
HID = 512
OUT1 = 256
OUT_CH = 128

F32 = jnp.float32
NC = 2
NS = 16
CH = 128
ROWS_PER_TILE = 640
E_PAD = 163840
NCHUNK = E_PAD // CH


def _proj_both_body(x_ref, dw1_ref, db1_ref, dlg_ref, dlb_ref, dw2_ref,
                    db2_ref, cw1t_ref, cb1_ref, clg_ref, clb_ref, cw2_ref,
                    cb2_ref, h2d_ref, dcs_ref, dcq_ref, h2c_ref, ccs_ref,
                    ccq_ref):
    i = pl.program_id(0)
    nsteps = pl.num_programs(0)
    x = x_ref[...]

    h = lax.dot_general(x, dw1_ref[...], (((1,), (1,)), ((), ())),
                        preferred_element_type=F32) + db1_ref[...]
    mu = jnp.mean(h, axis=-1, keepdims=True)
    v = jnp.mean((h - mu) ** 2, axis=-1, keepdims=True)
    h = (h - mu) / jnp.sqrt(v + 1e-5) * dlg_ref[...] + dlb_ref[...]
    h = jnp.maximum(h, 0.0)
    h2 = lax.dot_general(h, dw2_ref[...], (((1,), (1,)), ((), ())),
                         preferred_element_type=F32) + db2_ref[...]
    h2d_ref[...] = h2

    @pl.when(i == 0)
    def _():
        dcs_ref[...] = jnp.zeros_like(dcs_ref)
        dcq_ref[...] = jnp.zeros_like(dcq_ref)
        h2c_ref[...] = jnp.zeros_like(h2c_ref)

    dcs_ref[...] += jnp.sum(h2, axis=0, keepdims=True)
    dcq_ref[...] += jnp.sum(h2 * h2, axis=0, keepdims=True)

    h2c_ref[...] += lax.dot_general(
        x, cw1t_ref[...], (((0,), (0,)), ((), ())),
        preferred_element_type=F32)

    @pl.when(i == nsteps - 1)
    def _():
        hc = h2c_ref[...] + cb1_ref[...]
        muc = jnp.mean(hc, axis=-1, keepdims=True)
        vc = jnp.mean((hc - muc) ** 2, axis=-1, keepdims=True)
        hc = (hc - muc) / jnp.sqrt(vc + 1e-5) * clg_ref[...] + clb_ref[...]
        hc = jnp.maximum(hc, 0.0)
        h2c = lax.dot_general(hc, cw2_ref[...], (((1,), (1,)), ((), ())),
                              preferred_element_type=F32) + cb2_ref[...]
        h2c_ref[...] = h2c
        ccs_ref[...] = jnp.sum(h2c, axis=0, keepdims=True)
        ccq_ref[...] = jnp.sum(h2c * h2c, axis=0, keepdims=True)


def _proj_both(feats, dw1, db1, dlg, dlb, dw2, db2,
               cw1t, cb1, clg, clb, cw2, cb2):
    bm = 1000
    grid = (N_DRUG // bm,)
    vec = pl.BlockSpec((1, HID), lambda i: (0, 0))
    return pl.pallas_call(
        _proj_both_body,
        grid=grid,
        in_specs=[
            pl.BlockSpec((bm, TAIL), lambda i: (i, 0)),
            pl.BlockSpec((HID, TAIL), lambda i: (0, 0)),
            vec, vec, vec,
            pl.BlockSpec((HID, HID), lambda i: (0, 0)),
            vec,
            pl.BlockSpec((bm, HID), lambda i: (i, 0)),
            vec, vec, vec,
            pl.BlockSpec((HID, HID), lambda i: (0, 0)),
            vec,
        ],
        out_specs=[
            pl.BlockSpec((bm, HID), lambda i: (i, 0)),
            vec, vec,
            pl.BlockSpec((TAIL, HID), lambda i: (0, 0)),
            vec, vec,
        ],
        out_shape=[
            jax.ShapeDtypeStruct((N_DRUG, HID), F32),
            jax.ShapeDtypeStruct((1, HID), F32),
            jax.ShapeDtypeStruct((1, HID), F32),
            jax.ShapeDtypeStruct((TAIL, HID), F32),
            jax.ShapeDtypeStruct((1, HID), F32),
            jax.ShapeDtypeStruct((1, HID), F32),
        ],
    )(feats, dw1, db1, dlg, dlb, dw2, db2, cw1t, cb1, clg, clb, cw2, cb2)


def _bn_gcn1_body(h2_ref, cs_ref, cq_ref, g_ref, b_ref, p0_ref, p1_ref,
                  w_ref, ya_ref, yb_ref):
    i = pl.program_id(0)
    dom = i >= 8
    nrows = jnp.where(dom, float(TAIL), float(N_DRUG))
    cs = cs_ref[...]
    cq = cq_ref[...]
    mu = jnp.where(dom, cs[1:2, :], cs[0:1, :]) / nrows
    var = jnp.where(dom, cq[1:2, :], cq[0:1, :]) / nrows - mu * mu
    g = jnp.where(dom, g_ref[1:2, :], g_ref[0:1, :])
    b = jnp.where(dom, b_ref[1:2, :], b_ref[0:1, :])
    x = (h2_ref[...] - mu) / jnp.sqrt(var + 1e-5) * g + b
    x = jnp.maximum(x, 0.0)
    xw = lax.dot_general(x, w_ref[...], (((1,), (1,)), ((), ())),
                         preferred_element_type=F32)
    dis = lax.rsqrt(p0_ref[...] + p1_ref[...] + 1.0)
    y = xw * dis
    ya_ref[...] = y[:, :OUT_CH]
    yb_ref[...] = y[:, OUT_CH:]


def _bn_gcn1(h2, cs2, cq2, g2, b2, p0, p1, w):
    bm = 1000
    grid = (N // bm,)
    return pl.pallas_call(
        _bn_gcn1_body,
        grid=grid,
        in_specs=[
            pl.BlockSpec((bm, HID), lambda i: (i, 0)),
            pl.BlockSpec((2, HID), lambda i: (0, 0)),
            pl.BlockSpec((2, HID), lambda i: (0, 0)),
            pl.BlockSpec((2, HID), lambda i: (0, 0)),
            pl.BlockSpec((2, HID), lambda i: (0, 0)),
            pl.BlockSpec((bm, 1), lambda i: (i, 0)),
            pl.BlockSpec((bm, 1), lambda i: (i, 0)),
            pl.BlockSpec((OUT1, HID), lambda i: (0, 0)),
        ],
        out_specs=[
            pl.BlockSpec((bm, OUT_CH), lambda i: (i, 0)),
            pl.BlockSpec((bm, OUT_CH), lambda i: (i, 0)),
        ],
        out_shape=[
            jax.ShapeDtypeStruct((N, OUT_CH), F32),
            jax.ShapeDtypeStruct((N, OUT_CH), F32),
        ],
    )(h2, cs2, cq2, g2, b2, p0, p1, w)


def _gcn2_in_body(aa_ref, ab_ref, ya_ref, yb_ref, p0_ref, p1_ref, b1_ref,
                  w_ref, o_ref):
    dis = lax.rsqrt(p0_ref[...] + p1_ref[...] + 1.0)
    s = jnp.concatenate(
        [aa_ref[...].astype(F32) + ya_ref[...].astype(F32),
         ab_ref[...].astype(F32) + yb_ref[...].astype(F32)], axis=1)
    x1 = jnp.maximum(dis * s + b1_ref[...], 0.0)
    xw = lax.dot_general(x1, w_ref[...], (((1,), (1,)), ((), ())),
                         preferred_element_type=F32)
    o_ref[...] = xw * dis


def _gcn2_in(aggA, aggB, yA, yB, p0, p1, b1, w):
    bm = 1000
    grid = (N // bm,)
    return pl.pallas_call(
        _gcn2_in_body,
        grid=grid,
        in_specs=[
            pl.BlockSpec((bm, OUT_CH), lambda i: (i, 0)),
            pl.BlockSpec((bm, OUT_CH), lambda i: (i, 0)),
            pl.BlockSpec((bm, OUT_CH), lambda i: (i, 0)),
            pl.BlockSpec((bm, OUT_CH), lambda i: (i, 0)),
            pl.BlockSpec((bm, 1), lambda i: (i, 0)),
            pl.BlockSpec((bm, 1), lambda i: (i, 0)),
            pl.BlockSpec((1, OUT1), lambda i: (0, 0)),
            pl.BlockSpec((OUT_CH, OUT1), lambda i: (0, 0)),
        ],
        out_specs=pl.BlockSpec((bm, OUT_CH), lambda i: (i, 0)),
        out_shape=jax.ShapeDtypeStruct((N, OUT_CH), F32),
    )(aggA, aggB, yA, yB, p0, p1, b1, w)


def _final_body(a0_ref, a1_ref, y2_ref, p0_ref, p1_ref, b2_ref,
                lw_ref, lb_ref, df_ref, z_ref):
    dis = lax.rsqrt(p0_ref[...] + p1_ref[...] + 1.0)
    s = (a0_ref[...].astype(F32) + a1_ref[...].astype(F32)
         + y2_ref[...].astype(F32))
    df = dis * s + b2_ref[...]
    df_ref[...] = df
    z_ref[...] = lax.dot_general(df, lw_ref[...], (((1,), (1,)), ((), ())),
                                 preferred_element_type=F32) + lb_ref[...]


def _final(agg2P0, agg2P1, y2, p0, p1, b2, lw, lb):
    bm = 1000
    grid = (N // bm,)
    return pl.pallas_call(
        _final_body,
        grid=grid,
        in_specs=[
            pl.BlockSpec((bm, OUT_CH), lambda i: (i, 0)),
            pl.BlockSpec((bm, OUT_CH), lambda i: (i, 0)),
            pl.BlockSpec((bm, OUT_CH), lambda i: (i, 0)),
            pl.BlockSpec((bm, 1), lambda i: (i, 0)),
            pl.BlockSpec((bm, 1), lambda i: (i, 0)),
            pl.BlockSpec((1, OUT_CH), lambda i: (0, 0)),
            pl.BlockSpec((OUT_CH, OUT_CH), lambda i: (0, 0)),
            pl.BlockSpec((1, OUT_CH), lambda i: (0, 0)),
        ],
        out_specs=[
            pl.BlockSpec((bm, OUT_CH), lambda i: (i, 0)),
            pl.BlockSpec((bm, OUT_CH), lambda i: (i, 0)),
        ],
        out_shape=[
            jax.ShapeDtypeStruct((N, OUT_CH), F32),
            jax.ShapeDtypeStruct((N, OUT_CH), F32),
        ],
    )(agg2P0, agg2P1, y2, p0, p1, b2, lw, lb)


_MESH = plsc.VectorSubcoreMesh(core_axis_name="c", subcore_axis_name="s")


N_PAD = 10240


def _deg_body(dst_hbm, zer_hbm, one_hbm, out0, out1, acc, idxa, onev, sema):
    c = lax.axis_index("c")
    s = lax.axis_index("s")
    sl = pl.ds(640 * s, 640)
    nkt = NCHUNK // NC // NS
    t = s * NC + c
    pltpu.sync_copy(zer_hbm, acc.at[sl])
    pltpu.sync_copy(one_hbm, onev)
    pltpu.sync_copy(dst_hbm.at[t], idxa)
    plsc.subcore_barrier()

    @pl.loop(0, nkt)
    def _(k):
        pltpu.async_copy(onev, acc.at[idxa.at[k]], sema, add=True)

        @pl.when(k >= 3)
        def _():
            pltpu.make_async_copy(onev, acc.at[idxa.at[k - 3]], sema).wait()

    for t in range(3):
        pltpu.make_async_copy(onev, acc.at[idxa.at[nkt - 3 + t]], sema).wait()

    plsc.subcore_barrier()

    @pl.when(c == 0)
    def _():
        pltpu.sync_copy(acc.at[sl], out0.at[sl])

    @pl.when(c == 1)
    def _():
        pltpu.sync_copy(acc.at[sl], out1.at[sl])


def _deg_hist(dst2d, zeros640, ones128):
    nkt = NCHUNK // NC // NS
    fn = pl.kernel(
        _deg_body,
        out_type=(jax.ShapeDtypeStruct((N_PAD,), F32),
                  jax.ShapeDtypeStruct((N_PAD,), F32)),
        mesh=_MESH,
        scratch_types=[
            pltpu.VMEM_SHARED((N_PAD,), F32),
            pltpu.VMEM((nkt, CH), jnp.int32),
            pltpu.VMEM((CH,), F32),
            pltpu.SemaphoreType.DMA,
        ],
    )
    return fn(dst2d, zeros640, ones128)


PH = 40


def _make_seg_sum_body(split_cols, nphase, tile_fn):
    def body(ya_hbm, yb_hbm, src_hbm, dst_hbm, zer_hbm, outA, outB,
             acc, idxsa, idxda, rows4, semg, sema):
        c = lax.axis_index("c")
        s = lax.axis_index("s")
        sl = pl.ds(s * ROWS_PER_TILE, ROWS_PER_TILE)
        pltpu.sync_copy(zer_hbm, acc.at[sl])
        plsc.subcore_barrier()

        def fire_gather(k):
            b = lax.rem(k, 2)
            if split_cols:
                @pl.when(c == 0)
                def _():
                    pltpu.async_copy(ya_hbm.at[idxsa.at[k]],
                                     rows4.at[b], semg)

                @pl.when(c == 1)
                def _():
                    pltpu.async_copy(yb_hbm.at[idxsa.at[k]],
                                     rows4.at[b], semg)
            else:
                pltpu.async_copy(ya_hbm.at[idxsa.at[k]], rows4.at[b], semg)

        t = tile_fn(c, s)
        for h in range(nphase):
            pltpu.sync_copy(src_hbm.at[t, pl.ds(h * PH, PH)], idxsa)
            pltpu.sync_copy(dst_hbm.at[t, pl.ds(h * PH, PH)], idxda)
            fire_gather(0)

            @pl.loop(0, PH)
            def _(k):
                slot = lax.rem(k, 2)

                @pl.when(k >= 1)
                def _():
                    pltpu.make_async_copy(rows4.at[1 - slot],
                                          acc.at[idxda.at[k - 1]],
                                          sema).wait()

                @pl.when(k < PH - 1)
                def _():
                    fire_gather(k + 1)

                pltpu.make_async_copy(ya_hbm.at[idxsa.at[k]],
                                      rows4.at[slot], semg).wait()
                pltpu.async_copy(rows4.at[slot], acc.at[idxda.at[k]], sema,
                                 add=True)

            pltpu.make_async_copy(rows4.at[(PH - 1) % 2],
                                  acc.at[idxda.at[PH - 1]], sema).wait()

        plsc.subcore_barrier()

        @pl.when(c == 0)
        def _():
            pltpu.sync_copy(acc.at[sl], outA.at[sl])

        @pl.when(c == 1)
        def _():
            pltpu.sync_copy(acc.at[sl], outB.at[sl])

    return body


BF16 = jnp.bfloat16


def _seg_scratch():
    return [
        pltpu.VMEM_SHARED((N_PAD, OUT_CH), F32),
        pltpu.VMEM((PH, CH), jnp.int32),
        pltpu.VMEM((PH, CH), jnp.int32),
        pltpu.VMEM((2, CH, OUT_CH), F32),
        pltpu.SemaphoreType.DMA,
        pltpu.SemaphoreType.DMA,
    ]


def _seg_sum(yA, yB, src2d, dst2d, zeros_rows):
    body = _make_seg_sum_body(True, 2, lambda c, s: s)
    fn = pl.kernel(
        body,
        out_type=(jax.ShapeDtypeStruct((N_PAD, OUT_CH), F32),
                  jax.ShapeDtypeStruct((N_PAD, OUT_CH), F32)),
        mesh=_MESH,
        scratch_types=_seg_scratch(),
    )
    return fn(yA, yB, src2d, dst2d, zeros_rows)


def _seg_partial(y, src2d, dst2d, zeros_rows):
    body = _make_seg_sum_body(
        False, 1, lambda c, s: s * NC + c)
    fn = pl.kernel(
        body,
        out_type=(jax.ShapeDtypeStruct((N_PAD, OUT_CH), F32),
                  jax.ShapeDtypeStruct((N_PAD, OUT_CH), F32)),
        mesh=_MESH,
        scratch_types=_seg_scratch(),
    )
    return fn(y, y, src2d, dst2d, zeros_rows)


def _edge_body(z_hbm, src_hbm, dst_hbm, out, idxsa, idxda, buf3, semg, semw):
    c = lax.axis_index("c")
    s = lax.axis_index("s")
    w = s * NC + c
    nkt = jnp.where(w == NC * NS - 1, 1250 - 40 * (NC * NS - 1), 40)
    row0 = 40 * w
    pltpu.sync_copy(src_hbm.at[pl.ds(row0, 40)], idxsa)
    pltpu.sync_copy(dst_hbm.at[pl.ds(row0, 40)], idxda)

    def base(k):
        return pl.multiple_of((row0 + k) * CH, CH)

    def fire_gathers(k):
        slot = lax.rem(k, 3)
        pltpu.async_copy(z_hbm.at[idxsa.at[k]],
                         buf3.at[slot, :, pl.ds(0, OUT_CH)], semg)
        pltpu.async_copy(z_hbm.at[idxda.at[k]],
                         buf3.at[slot, :, pl.ds(OUT_CH, OUT_CH)], semg)

    fire_gathers(0)
    fire_gathers(1)

    @pl.loop(0, nkt)
    def _(k):
        slot = lax.rem(k, 3)

        @pl.when(k > 0)
        def _():
            pltpu.make_async_copy(buf3.at[lax.rem(k + 2, 3)],
                                  out.at[pl.ds(base(k - 1), CH)],
                                  semw).wait()

        @pl.when(k < nkt - 2)
        def _():
            fire_gathers(k + 2)

        pltpu.make_async_copy(z_hbm.at[idxsa.at[k]],
                              buf3.at[slot, :, pl.ds(0, OUT_CH)],
                              semg).wait()
        pltpu.make_async_copy(z_hbm.at[idxda.at[k]],
                              buf3.at[slot, :, pl.ds(OUT_CH, OUT_CH)],
                              semg).wait()
        pltpu.async_copy(buf3.at[slot], out.at[pl.ds(base(k), CH)], semw)

    pltpu.make_async_copy(buf3.at[lax.rem(nkt - 1, 3)],
                          out.at[pl.ds(base(nkt - 1), CH)], semw).wait()


def _edge_gather(z, src2d, dst2d):
    fn = pl.kernel(
        _edge_body,
        out_type=jax.ShapeDtypeStruct((E, 2 * OUT_CH), F32),
        mesh=_MESH,
        scratch_types=[
            pltpu.VMEM((40, CH), jnp.int32),
            pltpu.VMEM((40, CH), jnp.int32),
            pltpu.VMEM((3, CH, 2 * OUT_CH), F32),
            pltpu.SemaphoreType.DMA,
            pltpu.SemaphoreType.DMA,
        ],
    )
    return fn(z, src2d, dst2d)


def kernel(feats, dW1, db1, dlng, dlnb, dW2, db2, dbng, dbnb, cW1, cb1, clng,
           clnb, cW2, cb2, cbng, cbnb, convW1, convb1, convW2, convb2, linW,
           linb, edge_index, idx):
    src = edge_index[0]
    dst = edge_index[1]
    r = lambda v: v.reshape(1, -1)

    npad = E_PAD - E
    src_p = jnp.concatenate([src, jnp.zeros((npad,), jnp.int32)])
    pad_dst = N + (jnp.arange(npad, dtype=jnp.int32) % (N_PAD - N))
    dst_p = jnp.concatenate([dst, pad_dst])
    src2d = src_p.reshape(NCHUNK, CH)
    dst2d = dst_p.reshape(NCHUNK, CH)
    srcT16 = src_p.reshape(2 * PH, NS, CH).transpose(1, 0, 2)
    dstT16 = dst_p.reshape(2 * PH, NS, CH).transpose(1, 0, 2)
    srcT32 = src_p.reshape(PH, NC * NS, CH).transpose(1, 0, 2)
    dstT32 = dst_p.reshape(PH, NC * NS, CH).transpose(1, 0, 2)

    zeros640 = jnp.zeros((640,), F32)
    ones128 = jnp.ones((CH,), F32)
    d0, d1 = _deg_hist(dstT32, zeros640, ones128)
    p0 = d0.reshape(N_PAD, 1)
    p1 = d1.reshape(N_PAD, 1)

    (h2_d, cs_d, cq_d, h2_c, cs_c, cq_c) = _proj_both(
        feats, dW1, r(db1), r(dlng), r(dlnb), dW2, r(db2),
        cW1.T, r(cb1), r(clng), r(clnb), cW2, r(cb2))

    h2 = jnp.concatenate([h2_d, h2_c], axis=0)
    cs2 = jnp.concatenate([cs_d, cs_c], axis=0)
    cq2 = jnp.concatenate([cq_d, cq_c], axis=0)
    g2 = jnp.stack([dbng, cbng], axis=0)
    b2 = jnp.stack([dbnb, cbnb], axis=0)

    yA, yB = _bn_gcn1(h2, cs2, cq2, g2, b2, p0, p1, convW1)

    zrows128 = jnp.zeros((ROWS_PER_TILE, OUT_CH), F32)
    aggA, aggB = _seg_sum(yA, yB, srcT16, dstT16, zrows128)

    y2 = _gcn2_in(aggA, aggB, yA, yB, p0, p1, r(convb1), convW2)

    agg2P0, agg2P1 = _seg_partial(y2, srcT32, dstT32, zrows128)

    drug_f, z = _final(agg2P0, agg2P1, y2, p0, p1, r(convb2),
                       linW, r(linb))

    edge_feat = _edge_gather(z, src2d, dst2d)
    return (drug_f, edge_feat, idx)


# speedup vs baseline: 2.3967x; 1.5536x over previous
import functools

import jax
import jax.numpy as jnp
from jax import lax
from jax.experimental import pallas as pl
from jax.experimental.pallas import tpu as pltpu
from jax.experimental.pallas import tpu_sc as plsc

N_DRUG = 8000
TAIL = 2000
N = 10000
E = 160000
HID = 512
OUT1 = 256
OUT_CH = 128

F32 = jnp.float32
NC = 2
NS = 16
CH = 128
ROWS_PER_TILE = 640
E_PAD = 163840
NCHUNK = E_PAD // CH


def _proj_both_body(x_ref, dw1_ref, db1_ref, dlg_ref, dlb_ref, dw2_ref,
                    db2_ref, cw1t_ref, cb1_ref, clg_ref, clb_ref, cw2_ref,
                    cb2_ref, h2d_ref, dcs_ref, dcq_ref, h2c_ref, ccs_ref,
                    ccq_ref):
    i = pl.program_id(0)
    nsteps = pl.num_programs(0)
    x = x_ref[...]

    h = lax.dot_general(x, dw1_ref[...], (((1,), (1,)), ((), ())),
                        preferred_element_type=F32) + db1_ref[...]
    mu = jnp.mean(h, axis=-1, keepdims=True)
    v = jnp.mean((h - mu) ** 2, axis=-1, keepdims=True)
    h = (h - mu) / jnp.sqrt(v + 1e-5) * dlg_ref[...] + dlb_ref[...]
    h = jnp.maximum(h, 0.0)
    h2 = lax.dot_general(h, dw2_ref[...], (((1,), (1,)), ((), ())),
                         preferred_element_type=F32) + db2_ref[...]
    h2d_ref[...] = h2

    @pl.when(i == 0)
    def _():
        dcs_ref[...] = jnp.zeros_like(dcs_ref)
        dcq_ref[...] = jnp.zeros_like(dcq_ref)
        h2c_ref[...] = jnp.zeros_like(h2c_ref)

    dcs_ref[...] += jnp.sum(h2, axis=0, keepdims=True)
    dcq_ref[...] += jnp.sum(h2 * h2, axis=0, keepdims=True)

    h2c_ref[...] += lax.dot_general(
        x, cw1t_ref[...], (((0,), (0,)), ((), ())),
        preferred_element_type=F32)

    @pl.when(i == nsteps - 1)
    def _():
        hc = h2c_ref[...] + cb1_ref[...]
        muc = jnp.mean(hc, axis=-1, keepdims=True)
        vc = jnp.mean((hc - muc) ** 2, axis=-1, keepdims=True)
        hc = (hc - muc) / jnp.sqrt(vc + 1e-5) * clg_ref[...] + clb_ref[...]
        hc = jnp.maximum(hc, 0.0)
        h2c = lax.dot_general(hc, cw2_ref[...], (((1,), (1,)), ((), ())),
                              preferred_element_type=F32) + cb2_ref[...]
        h2c_ref[...] = h2c
        ccs_ref[...] = jnp.sum(h2c, axis=0, keepdims=True)
        ccq_ref[...] = jnp.sum(h2c * h2c, axis=0, keepdims=True)


def _proj_both(feats, dw1, db1, dlg, dlb, dw2, db2,
               cw1t, cb1, clg, clb, cw2, cb2):
    bm = 1000
    grid = (N_DRUG // bm,)
    vec = pl.BlockSpec((1, HID), lambda i: (0, 0))
    return pl.pallas_call(
        _proj_both_body,
        grid=grid,
        in_specs=[
            pl.BlockSpec((bm, TAIL), lambda i: (i, 0)),
            pl.BlockSpec((HID, TAIL), lambda i: (0, 0)),
            vec, vec, vec,
            pl.BlockSpec((HID, HID), lambda i: (0, 0)),
            vec,
            pl.BlockSpec((bm, HID), lambda i: (i, 0)),
            vec, vec, vec,
            pl.BlockSpec((HID, HID), lambda i: (0, 0)),
            vec,
        ],
        out_specs=[
            pl.BlockSpec((bm, HID), lambda i: (i, 0)),
            vec, vec,
            pl.BlockSpec((TAIL, HID), lambda i: (0, 0)),
            vec, vec,
        ],
        out_shape=[
            jax.ShapeDtypeStruct((N_DRUG, HID), F32),
            jax.ShapeDtypeStruct((1, HID), F32),
            jax.ShapeDtypeStruct((1, HID), F32),
            jax.ShapeDtypeStruct((TAIL, HID), F32),
            jax.ShapeDtypeStruct((1, HID), F32),
            jax.ShapeDtypeStruct((1, HID), F32),
        ],
    )(feats, dw1, db1, dlg, dlb, dw2, db2, cw1t, cb1, clg, clb, cw2, cb2)


def _bn_gcn1_body(h2_ref, cs_ref, cq_ref, g_ref, b_ref, p0_ref, p1_ref,
                  w_ref, ya_ref, yb_ref):
    i = pl.program_id(0)
    dom = i >= 8
    nrows = jnp.where(dom, float(TAIL), float(N_DRUG))
    cs = cs_ref[...]
    cq = cq_ref[...]
    mu = jnp.where(dom, cs[1:2, :], cs[0:1, :]) / nrows
    var = jnp.where(dom, cq[1:2, :], cq[0:1, :]) / nrows - mu * mu
    g = jnp.where(dom, g_ref[1:2, :], g_ref[0:1, :])
    b = jnp.where(dom, b_ref[1:2, :], b_ref[0:1, :])
    x = (h2_ref[...] - mu) / jnp.sqrt(var + 1e-5) * g + b
    x = jnp.maximum(x, 0.0)
    xw = lax.dot_general(x, w_ref[...], (((1,), (1,)), ((), ())),
                         preferred_element_type=F32)
    dis = lax.rsqrt(p0_ref[...] + p1_ref[...] + 1.0)
    y = xw * dis
    ya_ref[...] = y[:, :OUT_CH]
    yb_ref[...] = y[:, OUT_CH:]


def _bn_gcn1(h2, cs2, cq2, g2, b2, p0, p1, w):
    bm = 1000
    grid = (N // bm,)
    return pl.pallas_call(
        _bn_gcn1_body,
        grid=grid,
        in_specs=[
            pl.BlockSpec((bm, HID), lambda i: (i, 0)),
            pl.BlockSpec((2, HID), lambda i: (0, 0)),
            pl.BlockSpec((2, HID), lambda i: (0, 0)),
            pl.BlockSpec((2, HID), lambda i: (0, 0)),
            pl.BlockSpec((2, HID), lambda i: (0, 0)),
            pl.BlockSpec((bm, 1), lambda i: (i, 0)),
            pl.BlockSpec((bm, 1), lambda i: (i, 0)),
            pl.BlockSpec((OUT1, HID), lambda i: (0, 0)),
        ],
        out_specs=[
            pl.BlockSpec((bm, OUT_CH), lambda i: (i, 0)),
            pl.BlockSpec((bm, OUT_CH), lambda i: (i, 0)),
        ],
        out_shape=[
            jax.ShapeDtypeStruct((N, OUT_CH), F32),
            jax.ShapeDtypeStruct((N, OUT_CH), F32),
        ],
    )(h2, cs2, cq2, g2, b2, p0, p1, w)


def _gcn2_in_body(aa_ref, ab_ref, ya_ref, yb_ref, p0_ref, p1_ref, b1_ref,
                  w_ref, o_ref):
    dis = lax.rsqrt(p0_ref[...] + p1_ref[...] + 1.0)
    s = jnp.concatenate(
        [aa_ref[...].astype(F32) + ya_ref[...].astype(F32),
         ab_ref[...].astype(F32) + yb_ref[...].astype(F32)], axis=1)
    x1 = jnp.maximum(dis * s + b1_ref[...], 0.0)
    xw = lax.dot_general(x1, w_ref[...], (((1,), (1,)), ((), ())),
                         preferred_element_type=F32)
    o_ref[...] = xw * dis


def _gcn2_in(aggA, aggB, yA, yB, p0, p1, b1, w):
    bm = 1000
    grid = (N // bm,)
    return pl.pallas_call(
        _gcn2_in_body,
        grid=grid,
        in_specs=[
            pl.BlockSpec((bm, OUT_CH), lambda i: (i, 0)),
            pl.BlockSpec((bm, OUT_CH), lambda i: (i, 0)),
            pl.BlockSpec((bm, OUT_CH), lambda i: (i, 0)),
            pl.BlockSpec((bm, OUT_CH), lambda i: (i, 0)),
            pl.BlockSpec((bm, 1), lambda i: (i, 0)),
            pl.BlockSpec((bm, 1), lambda i: (i, 0)),
            pl.BlockSpec((1, OUT1), lambda i: (0, 0)),
            pl.BlockSpec((OUT_CH, OUT1), lambda i: (0, 0)),
        ],
        out_specs=pl.BlockSpec((bm, OUT_CH), lambda i: (i, 0)),
        out_shape=jax.ShapeDtypeStruct((N, OUT_CH), F32),
    )(aggA, aggB, yA, yB, p0, p1, b1, w)


def _final_body(a0_ref, a1_ref, y2_ref, p0_ref, p1_ref, b2_ref,
                lw_ref, lb_ref, df_ref, z_ref):
    dis = lax.rsqrt(p0_ref[...] + p1_ref[...] + 1.0)
    s = (a0_ref[...].astype(F32) + a1_ref[...].astype(F32)
         + y2_ref[...].astype(F32))
    df = dis * s + b2_ref[...]
    df_ref[...] = df
    z_ref[...] = lax.dot_general(df, lw_ref[...], (((1,), (1,)), ((), ())),
                                 preferred_element_type=F32) + lb_ref[...]


def _final(agg2P0, agg2P1, y2, p0, p1, b2, lw, lb):
    bm = 1000
    grid = (N // bm,)
    return pl.pallas_call(
        _final_body,
        grid=grid,
        in_specs=[
            pl.BlockSpec((bm, OUT_CH), lambda i: (i, 0)),
            pl.BlockSpec((bm, OUT_CH), lambda i: (i, 0)),
            pl.BlockSpec((bm, OUT_CH), lambda i: (i, 0)),
            pl.BlockSpec((bm, 1), lambda i: (i, 0)),
            pl.BlockSpec((bm, 1), lambda i: (i, 0)),
            pl.BlockSpec((1, OUT_CH), lambda i: (0, 0)),
            pl.BlockSpec((OUT_CH, OUT_CH), lambda i: (0, 0)),
            pl.BlockSpec((1, OUT_CH), lambda i: (0, 0)),
        ],
        out_specs=[
            pl.BlockSpec((bm, OUT_CH), lambda i: (i, 0)),
            pl.BlockSpec((bm, OUT_CH), lambda i: (i, 0)),
        ],
        out_shape=[
            jax.ShapeDtypeStruct((N, OUT_CH), F32),
            jax.ShapeDtypeStruct((N, OUT_CH), F32),
        ],
    )(agg2P0, agg2P1, y2, p0, p1, b2, lw, lb)


_MESH = plsc.VectorSubcoreMesh(core_axis_name="c", subcore_axis_name="s")


N_PAD = 10240


def _deg_body(dst_hbm, zer_hbm, one_hbm, d1_hbm, d2_hbm, d3_hbm, d4_hbm,
              d5_hbm, out0, out1, acc, idxa, onev, sema):
    c = lax.axis_index("c")
    s = lax.axis_index("s")
    sl = pl.ds(640 * s, 640)
    nkt = NCHUNK // NC // NS
    t = s * NC + c
    pltpu.sync_copy(zer_hbm, acc.at[sl])
    pltpu.sync_copy(one_hbm, onev)
    pltpu.sync_copy(dst_hbm.at[t], idxa)
    plsc.subcore_barrier()

    @pl.loop(0, nkt)
    def _(k):
        pltpu.async_copy(onev, acc.at[idxa.at[k]], sema, add=True)

        @pl.when(k >= 3)
        def _():
            pltpu.make_async_copy(onev, acc.at[idxa.at[k - 3]], sema).wait()

    for t in range(3):
        pltpu.make_async_copy(onev, acc.at[idxa.at[nkt - 3 + t]], sema).wait()

    plsc.subcore_barrier()

    @pl.when(c == 0)
    def _():
        pltpu.sync_copy(acc.at[sl], out0.at[sl])

    @pl.when(c == 1)
    def _():
        pltpu.sync_copy(acc.at[sl], out1.at[sl])


def _deg_hist(dst2d, zeros640, ones128, *layout_deps):
    nkt = NCHUNK // NC // NS
    fn = pl.kernel(
        _deg_body,
        out_type=(jax.ShapeDtypeStruct((N_PAD,), F32),
                  jax.ShapeDtypeStruct((N_PAD,), F32)),
        mesh=_MESH,
        scratch_types=[
            pltpu.VMEM_SHARED((N_PAD,), F32),
            pltpu.VMEM((nkt, CH), jnp.int32),
            pltpu.VMEM((CH,), F32),
            pltpu.SemaphoreType.DMA,
        ],
    )
    return fn(dst2d, zeros640, ones128, *layout_deps)


PH = 40


def _make_seg_sum_body(split_cols, nphase, tile_fn):
    def body(ya_hbm, yb_hbm, src_hbm, dst_hbm, zer_hbm, outA, outB,
             acc, idxsa, idxda, rows4, semg, sema):
        c = lax.axis_index("c")
        s = lax.axis_index("s")
        sl = pl.ds(s * ROWS_PER_TILE, ROWS_PER_TILE)
        pltpu.sync_copy(zer_hbm, acc.at[sl])
        plsc.subcore_barrier()

        def fire_gather(k):
            b = lax.rem(k, 2)
            if split_cols:
                @pl.when(c == 0)
                def _():
                    pltpu.async_copy(ya_hbm.at[idxsa.at[k]],
                                     rows4.at[b], semg)

                @pl.when(c == 1)
                def _():
                    pltpu.async_copy(yb_hbm.at[idxsa.at[k]],
                                     rows4.at[b], semg)
            else:
                pltpu.async_copy(ya_hbm.at[idxsa.at[k]], rows4.at[b], semg)

        t = tile_fn(c, s)
        for h in range(nphase):
            pltpu.sync_copy(src_hbm.at[t, pl.ds(h * PH, PH)], idxsa)
            pltpu.sync_copy(dst_hbm.at[t, pl.ds(h * PH, PH)], idxda)
            fire_gather(0)

            @pl.loop(0, PH)
            def _(k):
                slot = lax.rem(k, 2)

                @pl.when(k >= 1)
                def _():
                    pltpu.make_async_copy(rows4.at[1 - slot],
                                          acc.at[idxda.at[k - 1]],
                                          sema).wait()

                @pl.when(k < PH - 1)
                def _():
                    fire_gather(k + 1)

                pltpu.make_async_copy(ya_hbm.at[idxsa.at[k]],
                                      rows4.at[slot], semg).wait()
                pltpu.async_copy(rows4.at[slot], acc.at[idxda.at[k]], sema,
                                 add=True)

            pltpu.make_async_copy(rows4.at[(PH - 1) % 2],
                                  acc.at[idxda.at[PH - 1]], sema).wait()

        plsc.subcore_barrier()

        @pl.when(c == 0)
        def _():
            pltpu.sync_copy(acc.at[sl], outA.at[sl])

        @pl.when(c == 1)
        def _():
            pltpu.sync_copy(acc.at[sl], outB.at[sl])

    return body


BF16 = jnp.bfloat16


def _seg_scratch():
    return [
        pltpu.VMEM_SHARED((N_PAD, OUT_CH), F32),
        pltpu.VMEM((PH, CH), jnp.int32),
        pltpu.VMEM((PH, CH), jnp.int32),
        pltpu.VMEM((2, CH, OUT_CH), F32),
        pltpu.SemaphoreType.DMA,
        pltpu.SemaphoreType.DMA,
    ]


def _seg_sum(yA, yB, src2d, dst2d, zeros_rows):
    body = _make_seg_sum_body(True, 2, lambda c, s: s)
    fn = pl.kernel(
        body,
        out_type=(jax.ShapeDtypeStruct((N_PAD, OUT_CH), F32),
                  jax.ShapeDtypeStruct((N_PAD, OUT_CH), F32)),
        mesh=_MESH,
        scratch_types=_seg_scratch(),
    )
    return fn(yA, yB, src2d, dst2d, zeros_rows)


def _seg_partial(y, src2d, dst2d, zeros_rows):
    body = _make_seg_sum_body(
        False, 1, lambda c, s: s * NC + c)
    fn = pl.kernel(
        body,
        out_type=(jax.ShapeDtypeStruct((N_PAD, OUT_CH), F32),
                  jax.ShapeDtypeStruct((N_PAD, OUT_CH), F32)),
        mesh=_MESH,
        scratch_types=_seg_scratch(),
    )
    return fn(y, y, src2d, dst2d, zeros_rows)


def _edge_body(z_hbm, src_hbm, dst_hbm, out, idxsa, idxda, buf3, semg, semw):
    c = lax.axis_index("c")
    s = lax.axis_index("s")
    w = s * NC + c
    nkt = jnp.where(w == NC * NS - 1, 1250 - 40 * (NC * NS - 1), 40)
    row0 = 40 * w
    pltpu.sync_copy(src_hbm.at[pl.ds(row0, 40)], idxsa)
    pltpu.sync_copy(dst_hbm.at[pl.ds(row0, 40)], idxda)

    def base(k):
        return pl.multiple_of((row0 + k) * CH, CH)

    def fire_gathers(k):
        slot = lax.rem(k, 3)
        pltpu.async_copy(z_hbm.at[idxsa.at[k]],
                         buf3.at[slot, :, pl.ds(0, OUT_CH)], semg)
        pltpu.async_copy(z_hbm.at[idxda.at[k]],
                         buf3.at[slot, :, pl.ds(OUT_CH, OUT_CH)], semg)

    fire_gathers(0)
    fire_gathers(1)

    @pl.loop(0, nkt)
    def _(k):
        slot = lax.rem(k, 3)

        @pl.when(k > 0)
        def _():
            pltpu.make_async_copy(buf3.at[lax.rem(k + 2, 3)],
                                  out.at[pl.ds(base(k - 1), CH)],
                                  semw).wait()

        @pl.when(k < nkt - 2)
        def _():
            fire_gathers(k + 2)

        pltpu.make_async_copy(z_hbm.at[idxsa.at[k]],
                              buf3.at[slot, :, pl.ds(0, OUT_CH)],
                              semg).wait()
        pltpu.make_async_copy(z_hbm.at[idxda.at[k]],
                              buf3.at[slot, :, pl.ds(OUT_CH, OUT_CH)],
                              semg).wait()
        pltpu.async_copy(buf3.at[slot], out.at[pl.ds(base(k), CH)], semw)

    pltpu.make_async_copy(buf3.at[lax.rem(nkt - 1, 3)],
                          out.at[pl.ds(base(nkt - 1), CH)], semw).wait()


def _edge_gather(z, src2d, dst2d):
    fn = pl.kernel(
        _edge_body,
        out_type=jax.ShapeDtypeStruct((E, 2 * OUT_CH), F32),
        mesh=_MESH,
        scratch_types=[
            pltpu.VMEM((40, CH), jnp.int32),
            pltpu.VMEM((40, CH), jnp.int32),
            pltpu.VMEM((3, CH, 2 * OUT_CH), F32),
            pltpu.SemaphoreType.DMA,
            pltpu.SemaphoreType.DMA,
        ],
    )
    return fn(z, src2d, dst2d)


def kernel(feats, dW1, db1, dlng, dlnb, dW2, db2, dbng, dbnb, cW1, cb1, clng,
           clnb, cW2, cb2, cbng, cbnb, convW1, convb1, convW2, convb2, linW,
           linb, edge_index, idx):
    src = edge_index[0]
    dst = edge_index[1]
    r = lambda v: v.reshape(1, -1)

    npad = E_PAD - E
    pad_idx = jnp.arange(npad, dtype=jnp.int32)
    src_p = jnp.concatenate([src, pad_idx % N])
    pad_dst = N + (pad_idx % (N_PAD - N))
    dst_p = jnp.concatenate([dst, pad_dst])
    src2d = src_p.reshape(NCHUNK, CH)
    dst2d = dst_p.reshape(NCHUNK, CH)
    srcT16 = src_p.reshape(2 * PH, NS, CH).transpose(1, 0, 2)
    dstT16 = dst_p.reshape(2 * PH, NS, CH).transpose(1, 0, 2)
    srcT32 = src_p.reshape(PH, NC * NS, CH).transpose(1, 0, 2)
    dstT32 = dst_p.reshape(PH, NC * NS, CH).transpose(1, 0, 2)

    zeros640 = jnp.zeros((640,), F32)
    ones128 = jnp.ones((CH,), F32)
    d0, d1 = _deg_hist(dstT32, zeros640, ones128,
                       srcT16, dstT16, srcT32, src2d, dst2d)
    p0 = d0.reshape(N_PAD, 1)
    p1 = d1.reshape(N_PAD, 1)

    (h2_d, cs_d, cq_d, h2_c, cs_c, cq_c) = _proj_both(
        feats, dW1, r(db1), r(dlng), r(dlnb), dW2, r(db2),
        cW1.T, r(cb1), r(clng), r(clnb), cW2, r(cb2))

    h2 = jnp.concatenate([h2_d, h2_c], axis=0)
    cs2 = jnp.concatenate([cs_d, cs_c], axis=0)
    cq2 = jnp.concatenate([cq_d, cq_c], axis=0)
    g2 = jnp.stack([dbng, cbng], axis=0)
    b2 = jnp.stack([dbnb, cbnb], axis=0)

    yA, yB = _bn_gcn1(h2, cs2, cq2, g2, b2, p0, p1, convW1)

    zrows128 = jnp.zeros((ROWS_PER_TILE, OUT_CH), F32)
    aggA, aggB = _seg_sum(yA, yB, srcT16, dstT16, zrows128)

    y2 = _gcn2_in(aggA, aggB, yA, yB, p0, p1, r(convb1), convW2)

    agg2P0, agg2P1 = _seg_partial(y2, srcT32, dstT32, zrows128)

    drug_f, z = _final(agg2P0, agg2P1, y2, p0, p1, r(convb2),
                       linW, r(linb))

    edge_feat = _edge_gather(z, src2d, dst2d)
    return (drug_f, edge_feat, idx)


# speedup vs baseline: 2.4556x; 1.0246x over previous
import functools

import jax
import jax.numpy as jnp
from jax import lax
from jax.experimental import pallas as pl
from jax.experimental.pallas import tpu as pltpu
from jax.experimental.pallas import tpu_sc as plsc

N_DRUG = 8000
TAIL = 2000
N = 10000
E = 160000
HID = 512
OUT1 = 256
OUT_CH = 128

F32 = jnp.float32
NC = 2
NS = 16
CH = 128
ROWS_PER_TILE = 640
E_PAD = 163840
NCHUNK = E_PAD // CH


def _proj_both_body(x_ref, dw1_ref, db1_ref, dlg_ref, dlb_ref, dw2_ref,
                    db2_ref, cw1t_ref, cb1_ref, clg_ref, clb_ref, cw2_ref,
                    cb2_ref, h2d_ref, dcs_ref, dcq_ref, h2c_ref, ccs_ref,
                    ccq_ref):
    i = pl.program_id(0)
    nsteps = pl.num_programs(0)
    x = x_ref[...]

    h = lax.dot_general(x, dw1_ref[...], (((1,), (1,)), ((), ())),
                        preferred_element_type=F32) + db1_ref[...]
    mu = jnp.mean(h, axis=-1, keepdims=True)
    v = jnp.mean((h - mu) ** 2, axis=-1, keepdims=True)
    h = (h - mu) / jnp.sqrt(v + 1e-5) * dlg_ref[...] + dlb_ref[...]
    h = jnp.maximum(h, 0.0)
    h2 = lax.dot_general(h, dw2_ref[...], (((1,), (1,)), ((), ())),
                         preferred_element_type=F32) + db2_ref[...]
    h2d_ref[...] = h2

    @pl.when(i == 0)
    def _():
        dcs_ref[...] = jnp.zeros_like(dcs_ref)
        dcq_ref[...] = jnp.zeros_like(dcq_ref)
        h2c_ref[...] = jnp.zeros_like(h2c_ref)

    dcs_ref[...] += jnp.sum(h2, axis=0, keepdims=True)
    dcq_ref[...] += jnp.sum(h2 * h2, axis=0, keepdims=True)

    h2c_ref[...] += lax.dot_general(
        x, cw1t_ref[...], (((0,), (0,)), ((), ())),
        preferred_element_type=F32)

    @pl.when(i == nsteps - 1)
    def _():
        hc = h2c_ref[...] + cb1_ref[...]
        muc = jnp.mean(hc, axis=-1, keepdims=True)
        vc = jnp.mean((hc - muc) ** 2, axis=-1, keepdims=True)
        hc = (hc - muc) / jnp.sqrt(vc + 1e-5) * clg_ref[...] + clb_ref[...]
        hc = jnp.maximum(hc, 0.0)
        h2c = lax.dot_general(hc, cw2_ref[...], (((1,), (1,)), ((), ())),
                              preferred_element_type=F32) + cb2_ref[...]
        h2c_ref[...] = h2c
        ccs_ref[...] = jnp.sum(h2c, axis=0, keepdims=True)
        ccq_ref[...] = jnp.sum(h2c * h2c, axis=0, keepdims=True)


def _proj_both(feats, dw1, db1, dlg, dlb, dw2, db2,
               cw1t, cb1, clg, clb, cw2, cb2):
    bm = 1000
    grid = (N_DRUG // bm,)
    vec = pl.BlockSpec((1, HID), lambda i: (0, 0))
    return pl.pallas_call(
        _proj_both_body,
        grid=grid,
        in_specs=[
            pl.BlockSpec((bm, TAIL), lambda i: (i, 0)),
            pl.BlockSpec((HID, TAIL), lambda i: (0, 0)),
            vec, vec, vec,
            pl.BlockSpec((HID, HID), lambda i: (0, 0)),
            vec,
            pl.BlockSpec((bm, HID), lambda i: (i, 0)),
            vec, vec, vec,
            pl.BlockSpec((HID, HID), lambda i: (0, 0)),
            vec,
        ],
        out_specs=[
            pl.BlockSpec((bm, HID), lambda i: (i, 0)),
            vec, vec,
            pl.BlockSpec((TAIL, HID), lambda i: (0, 0)),
            vec, vec,
        ],
        out_shape=[
            jax.ShapeDtypeStruct((N_DRUG, HID), F32),
            jax.ShapeDtypeStruct((1, HID), F32),
            jax.ShapeDtypeStruct((1, HID), F32),
            jax.ShapeDtypeStruct((TAIL, HID), F32),
            jax.ShapeDtypeStruct((1, HID), F32),
            jax.ShapeDtypeStruct((1, HID), F32),
        ],
    )(feats, dw1, db1, dlg, dlb, dw2, db2, cw1t, cb1, clg, clb, cw2, cb2)


def _bn_gcn1_body(h2d_ref, h2c_ref, cs_ref, cq_ref, g_ref, b_ref, p0_ref,
                  p1_ref, w_ref, ya_ref, yb_ref):
    i = pl.program_id(0)
    dom = i >= 8
    nrows = jnp.where(dom, float(TAIL), float(N_DRUG))
    cs = cs_ref[...]
    cq = cq_ref[...]
    mu = jnp.where(dom, cs[1:2, :], cs[0:1, :]) / nrows
    var = jnp.where(dom, cq[1:2, :], cq[0:1, :]) / nrows - mu * mu
    g = jnp.where(dom, g_ref[1:2, :], g_ref[0:1, :])
    b = jnp.where(dom, b_ref[1:2, :], b_ref[0:1, :])
    h2 = jnp.where(dom, h2c_ref[...], h2d_ref[...])
    x = (h2 - mu) / jnp.sqrt(var + 1e-5) * g + b
    x = jnp.maximum(x, 0.0)
    xw = lax.dot_general(x, w_ref[...], (((1,), (1,)), ((), ())),
                         preferred_element_type=F32)
    dis = lax.rsqrt(p0_ref[...] + p1_ref[...] + 1.0)
    y = xw * dis
    ya_ref[...] = y[:, :OUT_CH]
    yb_ref[...] = y[:, OUT_CH:]


def _bn_gcn1(h2d, h2c, cs2, cq2, g2, b2, p0, p1, w):
    bm = 1000
    grid = (N // bm,)
    return pl.pallas_call(
        _bn_gcn1_body,
        grid=grid,
        in_specs=[
            pl.BlockSpec((bm, HID), lambda i: (jnp.minimum(i, 7), 0)),
            pl.BlockSpec((bm, HID), lambda i: (jnp.maximum(i - 8, 0), 0)),
            pl.BlockSpec((2, HID), lambda i: (0, 0)),
            pl.BlockSpec((2, HID), lambda i: (0, 0)),
            pl.BlockSpec((2, HID), lambda i: (0, 0)),
            pl.BlockSpec((2, HID), lambda i: (0, 0)),
            pl.BlockSpec((bm, 1), lambda i: (i, 0)),
            pl.BlockSpec((bm, 1), lambda i: (i, 0)),
            pl.BlockSpec((OUT1, HID), lambda i: (0, 0)),
        ],
        out_specs=[
            pl.BlockSpec((bm, OUT_CH), lambda i: (i, 0)),
            pl.BlockSpec((bm, OUT_CH), lambda i: (i, 0)),
        ],
        out_shape=[
            jax.ShapeDtypeStruct((N, OUT_CH), F32),
            jax.ShapeDtypeStruct((N, OUT_CH), F32),
        ],
    )(h2d, h2c, cs2, cq2, g2, b2, p0, p1, w)


def _gcn2_in_body(aa_ref, ab_ref, ya_ref, yb_ref, p0_ref, p1_ref, b1_ref,
                  w_ref, o_ref):
    dis = lax.rsqrt(p0_ref[...] + p1_ref[...] + 1.0)
    s = jnp.concatenate(
        [aa_ref[...].astype(F32) + ya_ref[...].astype(F32),
         ab_ref[...].astype(F32) + yb_ref[...].astype(F32)], axis=1)
    x1 = jnp.maximum(dis * s + b1_ref[...], 0.0)
    xw = lax.dot_general(x1, w_ref[...], (((1,), (1,)), ((), ())),
                         preferred_element_type=F32)
    o_ref[...] = xw * dis


def _gcn2_in(aggA, aggB, yA, yB, p0, p1, b1, w):
    bm = 1000
    grid = (N // bm,)
    return pl.pallas_call(
        _gcn2_in_body,
        grid=grid,
        in_specs=[
            pl.BlockSpec((bm, OUT_CH), lambda i: (i, 0)),
            pl.BlockSpec((bm, OUT_CH), lambda i: (i, 0)),
            pl.BlockSpec((bm, OUT_CH), lambda i: (i, 0)),
            pl.BlockSpec((bm, OUT_CH), lambda i: (i, 0)),
            pl.BlockSpec((bm, 1), lambda i: (i, 0)),
            pl.BlockSpec((bm, 1), lambda i: (i, 0)),
            pl.BlockSpec((1, OUT1), lambda i: (0, 0)),
            pl.BlockSpec((OUT_CH, OUT1), lambda i: (0, 0)),
        ],
        out_specs=pl.BlockSpec((bm, OUT_CH), lambda i: (i, 0)),
        out_shape=jax.ShapeDtypeStruct((N, OUT_CH), F32),
    )(aggA, aggB, yA, yB, p0, p1, b1, w)


def _final_body(a0_ref, a1_ref, y2_ref, p0_ref, p1_ref, b2_ref,
                lw_ref, lb_ref, df_ref, z_ref):
    dis = lax.rsqrt(p0_ref[...] + p1_ref[...] + 1.0)
    s = (a0_ref[...].astype(F32) + a1_ref[...].astype(F32)
         + y2_ref[...].astype(F32))
    df = dis * s + b2_ref[...]
    df_ref[...] = df
    z_ref[...] = lax.dot_general(df, lw_ref[...], (((1,), (1,)), ((), ())),
                                 preferred_element_type=F32) + lb_ref[...]


def _final(agg2P0, agg2P1, y2, p0, p1, b2, lw, lb):
    bm = 1000
    grid = (N // bm,)
    return pl.pallas_call(
        _final_body,
        grid=grid,
        in_specs=[
            pl.BlockSpec((bm, OUT_CH), lambda i: (i, 0)),
            pl.BlockSpec((bm, OUT_CH), lambda i: (i, 0)),
            pl.BlockSpec((bm, OUT_CH), lambda i: (i, 0)),
            pl.BlockSpec((bm, 1), lambda i: (i, 0)),
            pl.BlockSpec((bm, 1), lambda i: (i, 0)),
            pl.BlockSpec((1, OUT_CH), lambda i: (0, 0)),
            pl.BlockSpec((OUT_CH, OUT_CH), lambda i: (0, 0)),
            pl.BlockSpec((1, OUT_CH), lambda i: (0, 0)),
        ],
        out_specs=[
            pl.BlockSpec((bm, OUT_CH), lambda i: (i, 0)),
            pl.BlockSpec((bm, OUT_CH), lambda i: (i, 0)),
        ],
        out_shape=[
            jax.ShapeDtypeStruct((N, OUT_CH), F32),
            jax.ShapeDtypeStruct((N, OUT_CH), F32),
        ],
    )(agg2P0, agg2P1, y2, p0, p1, b2, lw, lb)


_MESH = plsc.VectorSubcoreMesh(core_axis_name="c", subcore_axis_name="s")


N_PAD = 10240


def _deg_body(dst_hbm, zer_hbm, one_hbm, d1_hbm, d2_hbm, d3_hbm, d4_hbm,
              d5_hbm, out0, out1, acc, idxa, onev, sema):
    c = lax.axis_index("c")
    s = lax.axis_index("s")
    sl = pl.ds(640 * s, 640)
    nkt = NCHUNK // NC // NS
    t = s * NC + c
    pltpu.sync_copy(zer_hbm, acc.at[sl])
    pltpu.sync_copy(one_hbm, onev)
    pltpu.sync_copy(dst_hbm.at[t], idxa)
    plsc.subcore_barrier()

    @pl.loop(0, nkt)
    def _(k):
        pltpu.async_copy(onev, acc.at[idxa.at[k]], sema, add=True)

        @pl.when(k >= 3)
        def _():
            pltpu.make_async_copy(onev, acc.at[idxa.at[k - 3]], sema).wait()

    for t in range(3):
        pltpu.make_async_copy(onev, acc.at[idxa.at[nkt - 3 + t]], sema).wait()

    plsc.subcore_barrier()

    @pl.when(c == 0)
    def _():
        pltpu.sync_copy(acc.at[sl], out0.at[sl])

    @pl.when(c == 1)
    def _():
        pltpu.sync_copy(acc.at[sl], out1.at[sl])


def _deg_hist(dst2d, zeros640, ones128, *layout_deps):
    nkt = NCHUNK // NC // NS
    fn = pl.kernel(
        _deg_body,
        out_type=(jax.ShapeDtypeStruct((N_PAD,), F32),
                  jax.ShapeDtypeStruct((N_PAD,), F32)),
        mesh=_MESH,
        scratch_types=[
            pltpu.VMEM_SHARED((N_PAD,), F32),
            pltpu.VMEM((nkt, CH), jnp.int32),
            pltpu.VMEM((CH,), F32),
            pltpu.SemaphoreType.DMA,
        ],
    )
    return fn(dst2d, zeros640, ones128, *layout_deps)


PH = 40


def _make_seg_sum_body(split_cols, nphase, tile_fn):
    def body(ya_hbm, yb_hbm, src_hbm, dst_hbm, zer_hbm, outA, outB,
             acc, idxsa, idxda, rows4, semg, sema):
        c = lax.axis_index("c")
        s = lax.axis_index("s")
        sl = pl.ds(s * ROWS_PER_TILE, ROWS_PER_TILE)
        pltpu.sync_copy(zer_hbm, acc.at[sl])
        plsc.subcore_barrier()

        def fire_gather(k):
            b = lax.rem(k, 2)
            if split_cols:
                @pl.when(c == 0)
                def _():
                    pltpu.async_copy(ya_hbm.at[idxsa.at[k]],
                                     rows4.at[b], semg)

                @pl.when(c == 1)
                def _():
                    pltpu.async_copy(yb_hbm.at[idxsa.at[k]],
                                     rows4.at[b], semg)
            else:
                pltpu.async_copy(ya_hbm.at[idxsa.at[k]], rows4.at[b], semg)

        t = tile_fn(c, s)
        for h in range(nphase):
            pltpu.sync_copy(src_hbm.at[t, pl.ds(h * PH, PH)], idxsa)
            pltpu.sync_copy(dst_hbm.at[t, pl.ds(h * PH, PH)], idxda)
            fire_gather(0)

            @pl.loop(0, PH)
            def _(k):
                slot = lax.rem(k, 2)

                @pl.when(k >= 1)
                def _():
                    pltpu.make_async_copy(rows4.at[1 - slot],
                                          acc.at[idxda.at[k - 1]],
                                          sema).wait()

                @pl.when(k < PH - 1)
                def _():
                    fire_gather(k + 1)

                pltpu.make_async_copy(ya_hbm.at[idxsa.at[k]],
                                      rows4.at[slot], semg).wait()
                pltpu.async_copy(rows4.at[slot], acc.at[idxda.at[k]], sema,
                                 add=True)

            pltpu.make_async_copy(rows4.at[(PH - 1) % 2],
                                  acc.at[idxda.at[PH - 1]], sema).wait()

        plsc.subcore_barrier()

        @pl.when(c == 0)
        def _():
            pltpu.sync_copy(acc.at[sl], outA.at[sl])

        @pl.when(c == 1)
        def _():
            pltpu.sync_copy(acc.at[sl], outB.at[sl])

    return body


BF16 = jnp.bfloat16


def _seg_scratch():
    return [
        pltpu.VMEM_SHARED((N_PAD, OUT_CH), F32),
        pltpu.VMEM((PH, CH), jnp.int32),
        pltpu.VMEM((PH, CH), jnp.int32),
        pltpu.VMEM((2, CH, OUT_CH), F32),
        pltpu.SemaphoreType.DMA,
        pltpu.SemaphoreType.DMA,
    ]


def _seg_sum(yA, yB, src2d, dst2d, zeros_rows):
    body = _make_seg_sum_body(True, 2, lambda c, s: s)
    fn = pl.kernel(
        body,
        out_type=(jax.ShapeDtypeStruct((N_PAD, OUT_CH), F32),
                  jax.ShapeDtypeStruct((N_PAD, OUT_CH), F32)),
        mesh=_MESH,
        scratch_types=_seg_scratch(),
    )
    return fn(yA, yB, src2d, dst2d, zeros_rows)


def _seg_partial(y, src2d, dst2d, zeros_rows):
    body = _make_seg_sum_body(
        False, 1, lambda c, s: s * NC + c)
    fn = pl.kernel(
        body,
        out_type=(jax.ShapeDtypeStruct((N_PAD, OUT_CH), F32),
                  jax.ShapeDtypeStruct((N_PAD, OUT_CH), F32)),
        mesh=_MESH,
        scratch_types=_seg_scratch(),
    )
    return fn(y, y, src2d, dst2d, zeros_rows)


def _edge_body(z_hbm, src_hbm, dst_hbm, out, idxsa, idxda, buf3, semg, semw):
    c = lax.axis_index("c")
    s = lax.axis_index("s")
    w = s * NC + c
    nkt = jnp.where(w == NC * NS - 1, 1250 - 40 * (NC * NS - 1), 40)
    row0 = 40 * w
    pltpu.sync_copy(src_hbm.at[pl.ds(row0, 40)], idxsa)
    pltpu.sync_copy(dst_hbm.at[pl.ds(row0, 40)], idxda)

    def base(k):
        return pl.multiple_of((row0 + k) * CH, CH)

    def fire_gathers(k):
        slot = lax.rem(k, 3)
        pltpu.async_copy(z_hbm.at[idxsa.at[k]],
                         buf3.at[slot, :, pl.ds(0, OUT_CH)], semg)
        pltpu.async_copy(z_hbm.at[idxda.at[k]],
                         buf3.at[slot, :, pl.ds(OUT_CH, OUT_CH)], semg)

    fire_gathers(0)
    fire_gathers(1)

    @pl.loop(0, nkt)
    def _(k):
        slot = lax.rem(k, 3)

        @pl.when(k > 0)
        def _():
            pltpu.make_async_copy(buf3.at[lax.rem(k + 2, 3)],
                                  out.at[pl.ds(base(k - 1), CH)],
                                  semw).wait()

        @pl.when(k < nkt - 2)
        def _():
            fire_gathers(k + 2)

        pltpu.make_async_copy(z_hbm.at[idxsa.at[k]],
                              buf3.at[slot, :, pl.ds(0, OUT_CH)],
                              semg).wait()
        pltpu.make_async_copy(z_hbm.at[idxda.at[k]],
                              buf3.at[slot, :, pl.ds(OUT_CH, OUT_CH)],
                              semg).wait()
        pltpu.async_copy(buf3.at[slot], out.at[pl.ds(base(k), CH)], semw)

    pltpu.make_async_copy(buf3.at[lax.rem(nkt - 1, 3)],
                          out.at[pl.ds(base(nkt - 1), CH)], semw).wait()


def _edge_gather(z, src2d, dst2d):
    fn = pl.kernel(
        _edge_body,
        out_type=jax.ShapeDtypeStruct((E, 2 * OUT_CH), F32),
        mesh=_MESH,
        scratch_types=[
            pltpu.VMEM((40, CH), jnp.int32),
            pltpu.VMEM((40, CH), jnp.int32),
            pltpu.VMEM((3, CH, 2 * OUT_CH), F32),
            pltpu.SemaphoreType.DMA,
            pltpu.SemaphoreType.DMA,
        ],
    )
    return fn(z, src2d, dst2d)


def kernel(feats, dW1, db1, dlng, dlnb, dW2, db2, dbng, dbnb, cW1, cb1, clng,
           clnb, cW2, cb2, cbng, cbnb, convW1, convb1, convW2, convb2, linW,
           linb, edge_index, idx):
    src = edge_index[0]
    dst = edge_index[1]
    r = lambda v: v.reshape(1, -1)

    npad = E_PAD - E
    pad_idx = jnp.arange(npad, dtype=jnp.int32)
    src_p = jnp.concatenate([src, pad_idx % N])
    pad_dst = N + (pad_idx % (N_PAD - N))
    dst_p = jnp.concatenate([dst, pad_dst])
    src2d = src_p.reshape(NCHUNK, CH)
    dst2d = dst_p.reshape(NCHUNK, CH)
    srcT16 = src_p.reshape(2 * PH, NS, CH).transpose(1, 0, 2)
    dstT16 = dst_p.reshape(2 * PH, NS, CH).transpose(1, 0, 2)
    srcT32 = src_p.reshape(PH, NC * NS, CH).transpose(1, 0, 2)
    dstT32 = dst_p.reshape(PH, NC * NS, CH).transpose(1, 0, 2)

    zeros640 = jnp.zeros((640,), F32)
    ones128 = jnp.ones((CH,), F32)
    d0, d1 = _deg_hist(dstT32, zeros640, ones128,
                       srcT16, dstT16, srcT32, src2d, dst2d)
    p0 = d0.reshape(N_PAD, 1)
    p1 = d1.reshape(N_PAD, 1)

    (h2_d, cs_d, cq_d, h2_c, cs_c, cq_c) = _proj_both(
        feats, dW1, r(db1), r(dlng), r(dlnb), dW2, r(db2),
        cW1.T, r(cb1), r(clng), r(clnb), cW2, r(cb2))

    cs2 = jnp.concatenate([cs_d, cs_c], axis=0)
    cq2 = jnp.concatenate([cq_d, cq_c], axis=0)
    g2 = jnp.stack([dbng, cbng], axis=0)
    b2 = jnp.stack([dbnb, cbnb], axis=0)

    yA, yB = _bn_gcn1(h2_d, h2_c, cs2, cq2, g2, b2, p0, p1, convW1)

    zrows128 = jnp.zeros((ROWS_PER_TILE, OUT_CH), F32)
    aggA, aggB = _seg_sum(yA, yB, srcT16, dstT16, zrows128)

    y2 = _gcn2_in(aggA, aggB, yA, yB, p0, p1, r(convb1), convW2)

    agg2P0, agg2P1 = _seg_partial(y2, srcT32, dstT32, zrows128)

    drug_f, z = _final(agg2P0, agg2P1, y2, p0, p1, r(convb2),
                       linW, r(linb))

    edge_feat = _edge_gather(z, src2d, dst2d)
    return (drug_f, edge_feat, idx)
